# Initial kernel scaffold; baseline (speedup 1.0000x reference)
import jax
import jax.numpy as jnp
from jax.experimental import pallas as pl


def kernel(pos, atom_features, edge_index, solvent_index, emb_solv, gamma_emb, Wm1, bm1, Wn1, bn1, Wm2, bm2, Wn2, bn2, Wm3, bm3, Wn3, bn3):
    raise NotImplementedError("write your pallas kernel here")


# speedup vs baseline: 1.1927x; 1.1927x over previous
import functools

import jax
import jax.numpy as jnp
import numpy as np
from jax import lax
from jax.experimental import pallas as pl
from jax.experimental.pallas import tpu as pltpu

N = 10000
E = 160000
H = 64
NK = 32
RADIUS = 0.6
FRACTION = 0.1
SCALING = 2.0
EPS_SOLVENT = 78.5

BLKE = 1280
EPAD = 163840
GRID_E = EPAD // BLKE
BLKN = 2000
GRID_N = N // BLKN
ACC_N = 10240
DUMMY = N

_M = 2.0 * (RADIUS - 0.1) / (NK + 1)
_C0 = 0.1 + _M
_CSTEP = ((RADIUS - _M) - (0.1 + _M)) / (NK - 1)
_FOURPI = float(4.0 * np.pi)


def _centers():
    ci = lax.broadcasted_iota(jnp.int32, (1, NK), 1).astype(jnp.float32)
    return _C0 + ci * _CSTEP


def _sig(x):
    return 1.0 / (1.0 + jnp.exp(-x))


def _silu(x):
    return x * _sig(x)


def _dsilu(x):
    s = _sig(x)
    return s * (1.0 + x * (1.0 - s))


def _ek_from_d(d):
    k = d - _centers()
    t = 1.0 - (k / _M) ** 2
    tm = jnp.maximum(t, 0.0)
    return tm * tm * tm


def _dek_dd(d):
    k = d - _centers()
    t = 1.0 - (k / _M) ** 2
    tm = jnp.maximum(t, 0.0)
    return 3.0 * tm * tm * (-2.0 * k / (_M * _M))


def _wfull(shape):
    return pl.BlockSpec(shape, lambda i: tuple(0 for _ in shape))


_VE = lambda i: (i, 0)
_VN = lambda i: (i, 0)


def _part_specs(w):
    return [pl.BlockSpec((1, BLKN, w), lambda i: (0, i, 0)),
            pl.BlockSpec((1, BLKN, w), lambda i: (1, i, 0))]


def _f1_body(t1s, t1d, wms, wmd, wmk, bm, m_ref, a_ref, geo_ref):
    s_ = t1s[...]
    d_ = t1d[...]
    diff = s_[:, 0:3] - d_[:, 0:3]
    d2 = jnp.sum(diff * diff, axis=1, keepdims=True) + 1e-12
    dd = jnp.sqrt(d2)
    ek = _ek_from_d(dd)
    a = (jnp.dot(s_[:, 3:6], wms[...]) + jnp.dot(d_[:, 3:6], wmd[...])
         + jnp.dot(ek, wmk[...]) + bm[...])
    a_ref[...] = a
    m_ref[...] = _silu(a)
    geo_ref[...] = jnp.concatenate(
        [dd, diff, jnp.zeros((BLKE, 4), jnp.float32)], axis=1)


def _f23_body(hs, hd, geo, wms, wmd, wmk, bm, m_ref, a_ref):
    ek = _ek_from_d(geo[...][:, 0:1])
    a = (jnp.dot(hs[...], wms[...]) + jnp.dot(hd[...], wmd[...])
         + jnp.dot(ek, wmk[...]) + bm[...])
    a_ref[...] = a
    m_ref[...] = _silu(a)


def _n1_body(aggA, aggB, t1, wn_agg, wn_x, wn_tok, bn, token, h_ref, hp_ref):
    agg = aggA[...][0] + aggB[...][0]
    tokc = jnp.dot(token[...], wn_tok[...]) + bn[...]
    hp = jnp.dot(agg, wn_agg[...]) + jnp.dot(t1[...][:, 3:6], wn_x[...]) + tokc
    hp_ref[...] = hp
    h_ref[...] = _silu(hp)


def _n2_body(aggA, aggB, x, wn_agg, wn_x, wn_tok, bn, token, h_ref, hp_ref):
    agg = aggA[...][0] + aggB[...][0]
    tokc = jnp.dot(token[...], wn_tok[...]) + bn[...]
    hp = jnp.dot(agg, wn_agg[...]) + jnp.dot(x[...], wn_x[...]) + tokc
    hp_ref[...] = hp
    h_ref[...] = _silu(hp)


def _n3gb_body(aggA, aggB, h2, t1, wn_agg, wn_x, wn_tok, bn, token, gamma,
               tb_ref, es_ref):
    agg = aggA[...][0] + aggB[...][0]
    tokc = jnp.dot(token[...], wn_tok[...]) + bn[...]
    c = jnp.dot(agg, wn_agg[...]) + jnp.dot(h2[...], wn_x[...]) + tokc
    q = t1[...][:, 3:4]
    sc0 = _sig(c[:, 0:1])
    sc1 = _sig(c[:, 1:2])
    B = 0.1 + 0.4 * sc1
    sa = FRACTION * sc0
    e_self = -0.5 * q * q / B * (1.0 - 1.0 / EPS_SOLVENT)
    e_sa = gamma[...][0, 0] * sa * _FOURPI * (B + 0.14) ** 2
    part = jnp.sum(e_self + e_sa)

    @pl.when(pl.program_id(0) == 0)
    def _():
        es_ref[...] = jnp.zeros_like(es_ref)

    es_ref[...] += jnp.reshape(part, (1, 1))
    tb_ref[...] = jnp.concatenate(
        [B, q, sc0, sc1, jnp.zeros((BLKN, 12), jnp.float32)], axis=1)


def _gbedge_body(tbs, tbd, geo, ep_ref, gdgb_ref, vsbs_ref, vsbd_ref):
    s_ = tbs[...]
    d_ = tbd[...]
    Bs = s_[:, 0:1]
    qs = s_[:, 1:2]
    Bd = d_[:, 0:1]
    qd = d_[:, 1:2]
    dd = geo[...][:, 0:1]
    d2 = dd * dd
    Bij = Bs * Bd
    u = jnp.exp(-d2 / (4.0 * Bij))
    fgb2 = d2 + Bij * u
    fgb = jnp.sqrt(fgb2)
    qq = qs * qd
    epair = -0.5 * qq / fgb
    rid = pl.program_id(0) * BLKE + lax.broadcasted_iota(jnp.int32, (BLKE, 1), 0)
    part = jnp.sum(jnp.where(rid < E, epair, 0.0))

    @pl.when(pl.program_id(0) == 0)
    def _():
        ep_ref[...] = jnp.zeros_like(ep_ref)

    ep_ref[...] += jnp.reshape(part, (1, 1))
    gfgb = 0.5 * qq / fgb2
    gdgb_ref[...] = gfgb * dd * (1.0 - 0.25 * u) / fgb
    gBij = gfgb * u * (1.0 + d2 / (4.0 * Bij)) / (2.0 * fgb)
    z = jnp.zeros((BLKE, 15), jnp.float32)
    vsbs_ref[...] = jnp.concatenate([gBij * Bd, z], axis=1)
    vsbd_ref[...] = jnp.concatenate([gBij * Bs, z], axis=1)


def _gbnode_body(tb, sbA, sbB, wn3_aggT, wn3_xT, gamma, gagg_ref, gxdir_ref):
    t_ = tb[...]
    B = t_[:, 0:1]
    q = t_[:, 1:2]
    sc0 = t_[:, 2:3]
    sc1 = t_[:, 3:4]
    g = gamma[...][0, 0]
    gB = (sbA[...][0][:, 0:1] + sbB[...][0][:, 0:1]
          + 0.5 * q * q / (B * B) * (1.0 - 1.0 / EPS_SOLVENT)
          + g * (FRACTION * sc0) * 2.0 * _FOURPI * (B + 0.14))
    gsa = g * _FOURPI * (B + 0.14) ** 2
    gc1 = gB * 0.4 * sc1 * (1.0 - sc1)
    gc0 = gsa * FRACTION * sc0 * (1.0 - sc0)
    Gc = jnp.concatenate([gc0, gc1], axis=1)
    gagg_ref[...] = jnp.dot(Gc, wn3_aggT[...])
    gxdir_ref[...] = jnp.dot(Gc, wn3_xT[...])


def _bedge3_body(gg, a, wmsT, wmdT, wmkT, vs_ref, vd_ref, gek_ref):
    Ga = gg[...] * _dsilu(a[...])
    vs_ref[...] = jnp.dot(Ga, wmsT[...])
    vd_ref[...] = jnp.dot(Ga, wmdT[...])
    gek_ref[...] = jnp.dot(Ga, wmkT[...])


def _bedge2_body(gg, a, gek_in, wmsT, wmdT, wmkT, vs_ref, vd_ref, gek_ref):
    Ga = gg[...] * _dsilu(a[...])
    vs_ref[...] = jnp.dot(Ga, wmsT[...])
    vd_ref[...] = jnp.dot(Ga, wmdT[...])
    gek_ref[...] = gek_in[...] + jnp.dot(Ga, wmkT[...])


def _bnode_body(gxdir, sxA, sxB, hp, wn_aggT, wn_xT, gagg_ref, gxn_ref):
    Gh = (gxdir[...] + sxA[...][0] + sxB[...][0]) * _dsilu(hp[...])
    gagg_ref[...] = jnp.dot(Gh, wn_aggT[...])
    gxn_ref[...] = jnp.dot(Gh, wn_xT[...])


def _b1node_body(gxdir, sxA, sxB, hp, wn_aggT, gagg_ref):
    Gh = (gxdir[...] + sxA[...][0] + sxB[...][0]) * _dsilu(hp[...])
    gagg_ref[...] = jnp.dot(Gh, wn_aggT[...])


def _bfinal_body(gg1, a1, gek23, geo, gdgb, wm1kT, fs_ref, fd_ref):
    Ga1 = gg1[...] * _dsilu(a1[...])
    gek = gek23[...] + jnp.dot(Ga1, wm1kT[...])
    g_ = geo[...]
    dd = g_[:, 0:1]
    diff = g_[:, 1:4]
    gd = gdgb[...] + jnp.sum(gek * _dek_dd(dd), axis=1, keepdims=True)
    f = (-gd / dd) * diff
    z = jnp.zeros((BLKE, 13), jnp.float32)
    fs_ref[...] = jnp.concatenate([f, z], axis=1)
    fd_ref[...] = jnp.concatenate([-f, z], axis=1)


def _fadd_body(pA, pB, out_ref):
    out_ref[...] = pA[...][0] + pB[...][0]


def _edge_spec(w):
    return pl.BlockSpec((BLKE, w), _VE)


def _node_spec(w):
    return pl.BlockSpec((BLKN, w), _VN)


def _eshape(w):
    return jax.ShapeDtypeStruct((EPAD, w), jnp.float32)


def _nshape(w):
    return jax.ShapeDtypeStruct((N, w), jnp.float32)


_SCALAR_SPEC = pl.BlockSpec((1, 1), lambda i: (0, 0))
_SCALAR_SHAPE = jax.ShapeDtypeStruct((1, 1), jnp.float32)


def _f1_call(t1s, t1d, wms, wmd, wmk, bm):
    return pl.pallas_call(
        _f1_body,
        grid=(GRID_E,),
        in_specs=[_edge_spec(16), _edge_spec(16), _wfull((3, H)), _wfull((3, H)),
                  _wfull((NK, H)), _wfull((1, H))],
        out_specs=[_edge_spec(H), _edge_spec(H), _edge_spec(8)],
        out_shape=[_eshape(H), _eshape(H), _eshape(8)],
    )(t1s, t1d, wms, wmd, wmk, bm)


def _f23_call(hs, hd, geo, wms, wmd, wmk, bm):
    return pl.pallas_call(
        _f23_body,
        grid=(GRID_E,),
        in_specs=[_edge_spec(H), _edge_spec(H), _edge_spec(8), _wfull((H, H)),
                  _wfull((H, H)), _wfull((NK, H)), _wfull((1, H))],
        out_specs=[_edge_spec(H), _edge_spec(H)],
        out_shape=[_eshape(H), _eshape(H)],
    )(hs, hd, geo, wms, wmd, wmk, bm)


def _n1_call(agg, t1, wn_agg, wn_x, wn_tok, bn, token):
    sA, sB = _part_specs(H)
    return pl.pallas_call(
        _n1_body,
        grid=(GRID_N,),
        in_specs=[sA, sB, _node_spec(16), _wfull((H, H)), _wfull((3, H)),
                  _wfull((H, H)), _wfull((1, H)), _wfull((1, H))],
        out_specs=[_node_spec(H), _node_spec(H)],
        out_shape=[_nshape(H), _nshape(H)],
    )(agg, agg, t1, wn_agg, wn_x, wn_tok, bn, token)


def _n2_call(agg, x, wn_agg, wn_x, wn_tok, bn, token):
    sA, sB = _part_specs(H)
    return pl.pallas_call(
        _n2_body,
        grid=(GRID_N,),
        in_specs=[sA, sB, _node_spec(H), _wfull((H, H)), _wfull((H, H)),
                  _wfull((H, H)), _wfull((1, H)), _wfull((1, H))],
        out_specs=[_node_spec(H), _node_spec(H)],
        out_shape=[_nshape(H), _nshape(H)],
    )(agg, agg, x, wn_agg, wn_x, wn_tok, bn, token)


def _n3gb_call(agg, h2, t1, wn_agg, wn_x, wn_tok, bn, token, gamma):
    sA, sB = _part_specs(H)
    return pl.pallas_call(
        _n3gb_body,
        grid=(GRID_N,),
        in_specs=[sA, sB, _node_spec(H), _node_spec(16), _wfull((H, 2)),
                  _wfull((H, 2)), _wfull((H, 2)), _wfull((1, 2)),
                  _wfull((1, H)), _wfull((1, 1))],
        out_specs=[_node_spec(16), _SCALAR_SPEC],
        out_shape=[_nshape(16), _SCALAR_SHAPE],
    )(agg, agg, h2, t1, wn_agg, wn_x, wn_tok, bn, token, gamma)


def _gbedge_call(tbs, tbd, geo):
    return pl.pallas_call(
        _gbedge_body,
        grid=(GRID_E,),
        in_specs=[_edge_spec(16), _edge_spec(16), _edge_spec(8)],
        out_specs=[_SCALAR_SPEC, _edge_spec(1), _edge_spec(16), _edge_spec(16)],
        out_shape=[_SCALAR_SHAPE, _eshape(1), _eshape(16), _eshape(16)],
    )(tbs, tbd, geo)


def _gbnode_call(tb, sb, wn3_aggT, wn3_xT, gamma):
    sA, sB = _part_specs(16)
    return pl.pallas_call(
        _gbnode_body,
        grid=(GRID_N,),
        in_specs=[_node_spec(16), sA, sB, _wfull((2, H)), _wfull((2, H)),
                  _wfull((1, 1))],
        out_specs=[_node_spec(H), _node_spec(H)],
        out_shape=[_nshape(H), _nshape(H)],
    )(tb, sb, sb, wn3_aggT, wn3_xT, gamma)


def _bedge3_call(gg, a, wmsT, wmdT, wmkT):
    return pl.pallas_call(
        _bedge3_body,
        grid=(GRID_E,),
        in_specs=[_edge_spec(H), _edge_spec(H), _wfull((H, H)), _wfull((H, H)),
                  _wfull((H, NK))],
        out_specs=[_edge_spec(H), _edge_spec(H), _edge_spec(NK)],
        out_shape=[_eshape(H), _eshape(H), _eshape(NK)],
    )(gg, a, wmsT, wmdT, wmkT)


def _bedge2_call(gg, a, gek_in, wmsT, wmdT, wmkT):
    return pl.pallas_call(
        _bedge2_body,
        grid=(GRID_E,),
        in_specs=[_edge_spec(H), _edge_spec(H), _edge_spec(NK), _wfull((H, H)),
                  _wfull((H, H)), _wfull((H, NK))],
        out_specs=[_edge_spec(H), _edge_spec(H), _edge_spec(NK)],
        out_shape=[_eshape(H), _eshape(H), _eshape(NK)],
    )(gg, a, gek_in, wmsT, wmdT, wmkT)


def _bnode_call(gxdir, sx, hp, wn_aggT, wn_xT):
    sA, sB = _part_specs(H)
    return pl.pallas_call(
        _bnode_body,
        grid=(GRID_N,),
        in_specs=[_node_spec(H), sA, sB, _node_spec(H), _wfull((H, H)),
                  _wfull((H, H))],
        out_specs=[_node_spec(H), _node_spec(H)],
        out_shape=[_nshape(H), _nshape(H)],
    )(gxdir, sx, sx, hp, wn_aggT, wn_xT)


def _b1node_call(gxdir, sx, hp, wn_aggT):
    sA, sB = _part_specs(H)
    return pl.pallas_call(
        _b1node_body,
        grid=(GRID_N,),
        in_specs=[_node_spec(H), sA, sB, _node_spec(H), _wfull((H, H))],
        out_specs=_node_spec(H),
        out_shape=_nshape(H),
    )(gxdir, sx, sx, hp, wn_aggT)


def _bfinal_call(gg1, a1, gek23, geo, gdgb, wm1kT):
    return pl.pallas_call(
        _bfinal_body,
        grid=(GRID_E,),
        in_specs=[_edge_spec(H), _edge_spec(H), _edge_spec(NK), _edge_spec(8),
                  _edge_spec(1), _wfull((H, NK))],
        out_specs=[_edge_spec(16), _edge_spec(16)],
        out_shape=[_eshape(16), _eshape(16)],
    )(gg1, a1, gek23, geo, gdgb, wm1kT)


def _fadd_call(sf):
    sA, sB = _part_specs(16)
    return pl.pallas_call(
        _fadd_body,
        grid=(GRID_N,),
        in_specs=[sA, sB],
        out_specs=_node_spec(16),
        out_shape=_nshape(16),
    )(sf, sf)


def _gather2(table, idx_s, idx_d):
    return table[idx_s], table[idx_d]


def _gather1(table, idx):
    return table[idx]


def _scatter2(vals_s, idx_s, vals_d, idx_d):
    w = vals_s.shape[1]
    p0 = jax.ops.segment_sum(vals_s, idx_s, num_segments=ACC_N)
    p1 = jax.ops.segment_sum(vals_d, idx_d, num_segments=ACC_N)
    return jnp.stack([p0, p1])


def _scatter1(vals, idx):
    w = vals.shape[1]
    p0 = jax.ops.segment_sum(vals, idx, num_segments=ACC_N)
    return jnp.stack([p0, jnp.zeros_like(p0)])


def kernel(pos, atom_features, edge_index, solvent_index, emb_solv, gamma_emb,
           Wm1, bm1, Wn1, bn1, Wm2, bm2, Wn2, bn2, Wm3, bm3, Wn3, bn3):
    src = edge_index[0]
    dst = edge_index[1]
    pad_g = jnp.zeros((EPAD - E,), jnp.int32)
    pad_s = jnp.full((EPAD - E,), DUMMY, jnp.int32)
    src_g = jnp.concatenate([src, pad_g])
    dst_g = jnp.concatenate([dst, pad_g])
    src_s = jnp.concatenate([src, pad_s])
    dst_s = jnp.concatenate([dst, pad_s])

    token = emb_solv[solvent_index[0]][None, :]
    gamma = gamma_emb[solvent_index[0], 0].reshape(1, 1)
    t1 = jnp.concatenate([pos, atom_features, jnp.zeros((N, 10), jnp.float32)],
                         axis=1)

    b1 = bm1[None, :]
    b2 = bm2[None, :]
    b3 = bm3[None, :]

    t1s, t1d = _gather2(t1, src_g, dst_g)
    m1, a1, geo = _f1_call(t1s, t1d, Wm1[0:3], Wm1[3:6], Wm1[6:38], b1)
    agg1 = _scatter1(m1, dst_s)
    h1, hp1 = _n1_call(agg1, t1, Wn1[0:64], Wn1[64:67], Wn1[67:131],
                       bn1[None, :], token)

    h1s, h1d = _gather2(h1, src_g, dst_g)
    m2, a2 = _f23_call(h1s, h1d, geo, Wm2[0:64], Wm2[64:128], Wm2[128:160], b2)
    agg2 = _scatter1(m2, dst_s)
    h2, hp2 = _n2_call(agg2, h1, Wn2[0:64], Wn2[64:128], Wn2[128:192],
                       bn2[None, :], token)

    h2s, h2d = _gather2(h2, src_g, dst_g)
    m3, a3 = _f23_call(h2s, h2d, geo, Wm3[0:64], Wm3[64:128], Wm3[128:160], b3)
    agg3 = _scatter1(m3, dst_s)
    tb, es_sum = _n3gb_call(agg3, h2, t1, Wn3[0:64], Wn3[64:128], Wn3[128:192],
                            bn3[None, :], token, gamma)

    tbs, tbd = _gather2(tb, src_g, dst_g)
    ep_sum, gdgb, vsbs, vsbd = _gbedge_call(tbs, tbd, geo)
    sb = _scatter2(vsbs, src_s, vsbd, dst_s)
    gagg3, gxdir3 = _gbnode_call(tb, sb, Wn3[0:64].T, Wn3[64:128].T, gamma)

    gg3 = _gather1(gagg3, dst_g)
    vs3, vd3, gek3 = _bedge3_call(gg3, a3, Wm3[0:64].T, Wm3[64:128].T,
                                  Wm3[128:160].T)
    sx3 = _scatter2(vs3, src_s, vd3, dst_s)
    gagg2, gxdir2 = _bnode_call(gxdir3, sx3, hp2, Wn2[0:64].T, Wn2[64:128].T)

    gg2 = _gather1(gagg2, dst_g)
    vs2, vd2, gek23 = _bedge2_call(gg2, a2, gek3, Wm2[0:64].T, Wm2[64:128].T,
                                   Wm2[128:160].T)
    sx2 = _scatter2(vs2, src_s, vd2, dst_s)
    gagg1 = _b1node_call(gxdir2, sx2, hp1, Wn1[0:64].T)

    gg1 = _gather1(gagg1, dst_g)
    fs, fd = _bfinal_call(gg1, a1, gek23, geo, gdgb, Wm1[6:38].T)
    sf = _scatter2(fs, src_s, fd, dst_s)
    fpad = _fadd_call(sf)

    forces = fpad[:, 0:3]
    energy = (ep_sum + es_sum).reshape(1, 1)
    return energy, forces


# speedup vs baseline: 1.9586x; 1.6422x over previous
import functools

import jax
import jax.numpy as jnp
import numpy as np
from jax import lax
from jax.experimental import pallas as pl
from jax.experimental.pallas import tpu as pltpu
from jax.experimental.pallas import tpu_sc as plsc

N = 10000
E = 160000
H = 64
NK = 32
RADIUS = 0.6
FRACTION = 0.1
SCALING = 2.0
EPS_SOLVENT = 78.5

BLKE = 1280
EPAD = 163840
GRID_E = EPAD // BLKE
BLKN = 2000
GRID_N = N // BLKN
ACC_N = 10240
DUMMY = N

_M = 2.0 * (RADIUS - 0.1) / (NK + 1)
_C0 = 0.1 + _M
_CSTEP = ((RADIUS - _M) - (0.1 + _M)) / (NK - 1)
_FOURPI = float(4.0 * np.pi)


def _centers():
    ci = lax.broadcasted_iota(jnp.int32, (1, NK), 1).astype(jnp.float32)
    return _C0 + ci * _CSTEP


def _sig(x):
    return 1.0 / (1.0 + jnp.exp(-x))


def _silu(x):
    return x * _sig(x)


def _dsilu(x):
    s = _sig(x)
    return s * (1.0 + x * (1.0 - s))


def _ek_from_d(d):
    k = d - _centers()
    t = 1.0 - (k / _M) ** 2
    tm = jnp.maximum(t, 0.0)
    return tm * tm * tm


def _dek_dd(d):
    k = d - _centers()
    t = 1.0 - (k / _M) ** 2
    tm = jnp.maximum(t, 0.0)
    return 3.0 * tm * tm * (-2.0 * k / (_M * _M))


def _wfull(shape):
    return pl.BlockSpec(shape, lambda i: tuple(0 for _ in shape))


_VE = lambda i: (i, 0)
_VN = lambda i: (i, 0)


def _part_specs(w):
    return [pl.BlockSpec((1, BLKN, w), lambda i: (0, i, 0)),
            pl.BlockSpec((1, BLKN, w), lambda i: (1, i, 0))]


def _f1_body(t1s, t1d, wms, wmd, wmk, bm, m_ref, a_ref, geo_ref):
    s_ = t1s[...]
    d_ = t1d[...]
    diff = s_[:, 0:3] - d_[:, 0:3]
    d2 = jnp.sum(diff * diff, axis=1, keepdims=True) + 1e-12
    dd = jnp.sqrt(d2)
    ek = _ek_from_d(dd)
    a = (jnp.dot(s_[:, 3:6], wms[...]) + jnp.dot(d_[:, 3:6], wmd[...])
         + jnp.dot(ek, wmk[...]) + bm[...])
    a_ref[...] = a
    m_ref[...] = _silu(a)
    geo_ref[...] = jnp.concatenate(
        [dd, diff, jnp.zeros((BLKE, 4), jnp.float32)], axis=1)


def _f23_body(hs, hd, geo, wms, wmd, wmk, bm, m_ref, a_ref):
    ek = _ek_from_d(geo[...][:, 0:1])
    a = (jnp.dot(hs[...], wms[...]) + jnp.dot(hd[...], wmd[...])
         + jnp.dot(ek, wmk[...]) + bm[...])
    a_ref[...] = a
    m_ref[...] = _silu(a)


def _n1_body(aggA, aggB, t1, wn_agg, wn_x, wn_tok, bn, token, h_ref, hp_ref):
    agg = aggA[...][0] + aggB[...][0]
    tokc = jnp.dot(token[...], wn_tok[...]) + bn[...]
    hp = jnp.dot(agg, wn_agg[...]) + jnp.dot(t1[...][:, 3:6], wn_x[...]) + tokc
    hp_ref[...] = hp
    h_ref[...] = _silu(hp)


def _n2_body(aggA, aggB, x, wn_agg, wn_x, wn_tok, bn, token, h_ref, hp_ref):
    agg = aggA[...][0] + aggB[...][0]
    tokc = jnp.dot(token[...], wn_tok[...]) + bn[...]
    hp = jnp.dot(agg, wn_agg[...]) + jnp.dot(x[...], wn_x[...]) + tokc
    hp_ref[...] = hp
    h_ref[...] = _silu(hp)


def _n3gb_body(aggA, aggB, h2, t1, wn_agg, wn_x, wn_tok, bn, token, gamma,
               tb_ref, es_ref):
    agg = aggA[...][0] + aggB[...][0]
    tokc = jnp.dot(token[...], wn_tok[...]) + bn[...]
    c = jnp.dot(agg, wn_agg[...]) + jnp.dot(h2[...], wn_x[...]) + tokc
    q = t1[...][:, 3:4]
    sc0 = _sig(c[:, 0:1])
    sc1 = _sig(c[:, 1:2])
    B = 0.1 + 0.4 * sc1
    sa = FRACTION * sc0
    e_self = -0.5 * q * q / B * (1.0 - 1.0 / EPS_SOLVENT)
    e_sa = gamma[...][0, 0] * sa * _FOURPI * (B + 0.14) ** 2
    part = jnp.sum(e_self + e_sa)

    @pl.when(pl.program_id(0) == 0)
    def _():
        es_ref[...] = jnp.zeros_like(es_ref)

    es_ref[...] += jnp.reshape(part, (1, 1))
    tb_ref[...] = jnp.concatenate(
        [B, q, sc0, sc1, jnp.zeros((BLKN, 12), jnp.float32)], axis=1)


def _gbedge_body(tbs, tbd, geo, ep_ref, gdgb_ref, vsbs_ref, vsbd_ref):
    s_ = tbs[...]
    d_ = tbd[...]
    Bs = s_[:, 0:1]
    qs = s_[:, 1:2]
    Bd = d_[:, 0:1]
    qd = d_[:, 1:2]
    dd = geo[...][:, 0:1]
    d2 = dd * dd
    Bij = Bs * Bd
    u = jnp.exp(-d2 / (4.0 * Bij))
    fgb2 = d2 + Bij * u
    fgb = jnp.sqrt(fgb2)
    qq = qs * qd
    epair = -0.5 * qq / fgb
    rid = pl.program_id(0) * BLKE + lax.broadcasted_iota(jnp.int32, (BLKE, 1), 0)
    part = jnp.sum(jnp.where(rid < E, epair, 0.0))

    @pl.when(pl.program_id(0) == 0)
    def _():
        ep_ref[...] = jnp.zeros_like(ep_ref)

    ep_ref[...] += jnp.reshape(part, (1, 1))
    gfgb = 0.5 * qq / fgb2
    gdgb_ref[...] = gfgb * dd * (1.0 - 0.25 * u) / fgb
    gBij = gfgb * u * (1.0 + d2 / (4.0 * Bij)) / (2.0 * fgb)
    z = jnp.zeros((BLKE, 15), jnp.float32)
    vsbs_ref[...] = jnp.concatenate([gBij * Bd, z], axis=1)
    vsbd_ref[...] = jnp.concatenate([gBij * Bs, z], axis=1)


def _gbnode_body(tb, sbA, sbB, wn3_aggT, wn3_xT, gamma, gagg_ref, gxdir_ref):
    t_ = tb[...]
    B = t_[:, 0:1]
    q = t_[:, 1:2]
    sc0 = t_[:, 2:3]
    sc1 = t_[:, 3:4]
    g = gamma[...][0, 0]
    gB = (sbA[...][0][:, 0:1] + sbB[...][0][:, 0:1]
          + 0.5 * q * q / (B * B) * (1.0 - 1.0 / EPS_SOLVENT)
          + g * (FRACTION * sc0) * 2.0 * _FOURPI * (B + 0.14))
    gsa = g * _FOURPI * (B + 0.14) ** 2
    gc1 = gB * 0.4 * sc1 * (1.0 - sc1)
    gc0 = gsa * FRACTION * sc0 * (1.0 - sc0)
    Gc = jnp.concatenate([gc0, gc1], axis=1)
    gagg_ref[...] = jnp.dot(Gc, wn3_aggT[...])
    gxdir_ref[...] = jnp.dot(Gc, wn3_xT[...])


def _bedge3_body(gg, a, wmsT, wmdT, wmkT, vs_ref, vd_ref, gek_ref):
    Ga = gg[...] * _dsilu(a[...])
    vs_ref[...] = jnp.dot(Ga, wmsT[...])
    vd_ref[...] = jnp.dot(Ga, wmdT[...])
    gek_ref[...] = jnp.dot(Ga, wmkT[...])


def _bedge2_body(gg, a, gek_in, wmsT, wmdT, wmkT, vs_ref, vd_ref, gek_ref):
    Ga = gg[...] * _dsilu(a[...])
    vs_ref[...] = jnp.dot(Ga, wmsT[...])
    vd_ref[...] = jnp.dot(Ga, wmdT[...])
    gek_ref[...] = gek_in[...] + jnp.dot(Ga, wmkT[...])


def _bnode_body(gxdir, sxA, sxB, hp, wn_aggT, wn_xT, gagg_ref, gxn_ref):
    Gh = (gxdir[...] + sxA[...][0] + sxB[...][0]) * _dsilu(hp[...])
    gagg_ref[...] = jnp.dot(Gh, wn_aggT[...])
    gxn_ref[...] = jnp.dot(Gh, wn_xT[...])


def _b1node_body(gxdir, sxA, sxB, hp, wn_aggT, gagg_ref):
    Gh = (gxdir[...] + sxA[...][0] + sxB[...][0]) * _dsilu(hp[...])
    gagg_ref[...] = jnp.dot(Gh, wn_aggT[...])


def _bfinal_body(gg1, a1, gek23, geo, gdgb, wm1kT, fs_ref, fd_ref):
    Ga1 = gg1[...] * _dsilu(a1[...])
    gek = gek23[...] + jnp.dot(Ga1, wm1kT[...])
    g_ = geo[...]
    dd = g_[:, 0:1]
    diff = g_[:, 1:4]
    gd = gdgb[...] + jnp.sum(gek * _dek_dd(dd), axis=1, keepdims=True)
    f = (-gd / dd) * diff
    z = jnp.zeros((BLKE, 13), jnp.float32)
    fs_ref[...] = jnp.concatenate([f, z], axis=1)
    fd_ref[...] = jnp.concatenate([-f, z], axis=1)


def _fadd_body(pA, pB, out_ref):
    out_ref[...] = pA[...][0] + pB[...][0]


def _edge_spec(w):
    return pl.BlockSpec((BLKE, w), _VE)


def _node_spec(w):
    return pl.BlockSpec((BLKN, w), _VN)


def _eshape(w):
    return jax.ShapeDtypeStruct((EPAD, w), jnp.float32)


def _nshape(w):
    return jax.ShapeDtypeStruct((N, w), jnp.float32)


_SCALAR_SPEC = pl.BlockSpec((1, 1), lambda i: (0, 0))
_SCALAR_SHAPE = jax.ShapeDtypeStruct((1, 1), jnp.float32)


def _f1_call(t1s, t1d, wms, wmd, wmk, bm):
    return pl.pallas_call(
        _f1_body,
        grid=(GRID_E,),
        in_specs=[_edge_spec(16), _edge_spec(16), _wfull((3, H)), _wfull((3, H)),
                  _wfull((NK, H)), _wfull((1, H))],
        out_specs=[_edge_spec(H), _edge_spec(H), _edge_spec(8)],
        out_shape=[_eshape(H), _eshape(H), _eshape(8)],
    )(t1s, t1d, wms, wmd, wmk, bm)


def _f23_call(hs, hd, geo, wms, wmd, wmk, bm):
    return pl.pallas_call(
        _f23_body,
        grid=(GRID_E,),
        in_specs=[_edge_spec(H), _edge_spec(H), _edge_spec(8), _wfull((H, H)),
                  _wfull((H, H)), _wfull((NK, H)), _wfull((1, H))],
        out_specs=[_edge_spec(H), _edge_spec(H)],
        out_shape=[_eshape(H), _eshape(H)],
    )(hs, hd, geo, wms, wmd, wmk, bm)


def _n1_call(agg, t1, wn_agg, wn_x, wn_tok, bn, token):
    sA, sB = _part_specs(H)
    return pl.pallas_call(
        _n1_body,
        grid=(GRID_N,),
        in_specs=[sA, sB, _node_spec(16), _wfull((H, H)), _wfull((3, H)),
                  _wfull((H, H)), _wfull((1, H)), _wfull((1, H))],
        out_specs=[_node_spec(H), _node_spec(H)],
        out_shape=[_nshape(H), _nshape(H)],
    )(agg, agg, t1, wn_agg, wn_x, wn_tok, bn, token)


def _n2_call(agg, x, wn_agg, wn_x, wn_tok, bn, token):
    sA, sB = _part_specs(H)
    return pl.pallas_call(
        _n2_body,
        grid=(GRID_N,),
        in_specs=[sA, sB, _node_spec(H), _wfull((H, H)), _wfull((H, H)),
                  _wfull((H, H)), _wfull((1, H)), _wfull((1, H))],
        out_specs=[_node_spec(H), _node_spec(H)],
        out_shape=[_nshape(H), _nshape(H)],
    )(agg, agg, x, wn_agg, wn_x, wn_tok, bn, token)


def _n3gb_call(agg, h2, t1, wn_agg, wn_x, wn_tok, bn, token, gamma):
    sA, sB = _part_specs(H)
    return pl.pallas_call(
        _n3gb_body,
        grid=(GRID_N,),
        in_specs=[sA, sB, _node_spec(H), _node_spec(16), _wfull((H, 2)),
                  _wfull((H, 2)), _wfull((H, 2)), _wfull((1, 2)),
                  _wfull((1, H)), _wfull((1, 1))],
        out_specs=[_node_spec(16), _SCALAR_SPEC],
        out_shape=[_nshape(16), _SCALAR_SHAPE],
    )(agg, agg, h2, t1, wn_agg, wn_x, wn_tok, bn, token, gamma)


def _gbedge_call(tbs, tbd, geo):
    return pl.pallas_call(
        _gbedge_body,
        grid=(GRID_E,),
        in_specs=[_edge_spec(16), _edge_spec(16), _edge_spec(8)],
        out_specs=[_SCALAR_SPEC, _edge_spec(1), _edge_spec(16), _edge_spec(16)],
        out_shape=[_SCALAR_SHAPE, _eshape(1), _eshape(16), _eshape(16)],
    )(tbs, tbd, geo)


def _gbnode_call(tb, sb, wn3_aggT, wn3_xT, gamma):
    sA, sB = _part_specs(16)
    return pl.pallas_call(
        _gbnode_body,
        grid=(GRID_N,),
        in_specs=[_node_spec(16), sA, sB, _wfull((2, H)), _wfull((2, H)),
                  _wfull((1, 1))],
        out_specs=[_node_spec(H), _node_spec(H)],
        out_shape=[_nshape(H), _nshape(H)],
    )(tb, sb, sb, wn3_aggT, wn3_xT, gamma)


def _bedge3_call(gg, a, wmsT, wmdT, wmkT):
    return pl.pallas_call(
        _bedge3_body,
        grid=(GRID_E,),
        in_specs=[_edge_spec(H), _edge_spec(H), _wfull((H, H)), _wfull((H, H)),
                  _wfull((H, NK))],
        out_specs=[_edge_spec(H), _edge_spec(H), _edge_spec(NK)],
        out_shape=[_eshape(H), _eshape(H), _eshape(NK)],
    )(gg, a, wmsT, wmdT, wmkT)


def _bedge2_call(gg, a, gek_in, wmsT, wmdT, wmkT):
    return pl.pallas_call(
        _bedge2_body,
        grid=(GRID_E,),
        in_specs=[_edge_spec(H), _edge_spec(H), _edge_spec(NK), _wfull((H, H)),
                  _wfull((H, H)), _wfull((H, NK))],
        out_specs=[_edge_spec(H), _edge_spec(H), _edge_spec(NK)],
        out_shape=[_eshape(H), _eshape(H), _eshape(NK)],
    )(gg, a, gek_in, wmsT, wmdT, wmkT)


def _bnode_call(gxdir, sx, hp, wn_aggT, wn_xT):
    sA, sB = _part_specs(H)
    return pl.pallas_call(
        _bnode_body,
        grid=(GRID_N,),
        in_specs=[_node_spec(H), sA, sB, _node_spec(H), _wfull((H, H)),
                  _wfull((H, H))],
        out_specs=[_node_spec(H), _node_spec(H)],
        out_shape=[_nshape(H), _nshape(H)],
    )(gxdir, sx, sx, hp, wn_aggT, wn_xT)


def _b1node_call(gxdir, sx, hp, wn_aggT):
    sA, sB = _part_specs(H)
    return pl.pallas_call(
        _b1node_body,
        grid=(GRID_N,),
        in_specs=[_node_spec(H), sA, sB, _node_spec(H), _wfull((H, H))],
        out_specs=_node_spec(H),
        out_shape=_nshape(H),
    )(gxdir, sx, sx, hp, wn_aggT)


def _bfinal_call(gg1, a1, gek23, geo, gdgb, wm1kT):
    return pl.pallas_call(
        _bfinal_body,
        grid=(GRID_E,),
        in_specs=[_edge_spec(H), _edge_spec(H), _edge_spec(NK), _edge_spec(8),
                  _edge_spec(1), _wfull((H, NK))],
        out_specs=[_edge_spec(16), _edge_spec(16)],
        out_shape=[_eshape(16), _eshape(16)],
    )(gg1, a1, gek23, geo, gdgb, wm1kT)


def _fadd_call(sf):
    sA, sB = _part_specs(16)
    return pl.pallas_call(
        _fadd_body,
        grid=(GRID_N,),
        in_specs=[sA, sB],
        out_specs=_node_spec(16),
        out_shape=_nshape(16),
    )(sf, sf)


_KF = 8
_CH = 128
_NW = 32
_RPT = ACC_N // 16


def _sc_mesh():
    return plsc.VectorSubcoreMesh(core_axis_name="c", subcore_axis_name="s")


_SC_PARAMS = pltpu.CompilerParams(use_tc_tiling_on_sc=False)


@functools.lru_cache(maxsize=None)
def _mk_gather2(w, nt):
    rw = EPAD // 16
    k = rw // _CH
    ng = k // _KF

    @functools.partial(
        pl.kernel,
        out_type=[jax.ShapeDtypeStruct((EPAD, w), jnp.float32),
                  jax.ShapeDtypeStruct((EPAD, w), jnp.float32)],
        mesh=_sc_mesh(),
        compiler_params=_SC_PARAMS,
        scratch_types=[pltpu.VMEM((k, _CH), jnp.int32),
                       pltpu.VMEM((_KF * _CH, w), jnp.float32),
                       pltpu.SemaphoreType.DMA],
    )
    def kern(table, idxs, idxd, outs, outd, idx_v, buf, sem):
        c = lax.axis_index("c")
        s = lax.axis_index("s")
        wid = s * 2 + c
        lw = wid % 16

        def process(idx_hbm, out_hbm):
            pltpu.sync_copy(idx_hbm.at[pl.ds(lw * k, k)], idx_v)

            def grp(g, _):
                descs = [
                    pltpu.async_copy(table.at[idx_v.at[g * _KF + j]],
                                     buf.at[pl.ds(j * _CH, _CH)], sem)
                    for j in range(_KF)
                ]
                for dsc in descs:
                    dsc.wait()
                pltpu.sync_copy(
                    buf, out_hbm.at[pl.ds(lw * rw + g * _KF * _CH, _KF * _CH)])
                return 0

            lax.fori_loop(0, ng, grp, 0)

        @pl.when(wid < 16)
        def _():
            process(idxs, outs)

        @pl.when(wid >= 16)
        def _():
            process(idxd, outd)

    return kern


@functools.lru_cache(maxsize=None)
def _mk_gather1(w, nt):
    rw = EPAD // _NW
    k = rw // _CH
    ng = k // _KF

    @functools.partial(
        pl.kernel,
        out_type=jax.ShapeDtypeStruct((EPAD, w), jnp.float32),
        mesh=_sc_mesh(),
        compiler_params=_SC_PARAMS,
        scratch_types=[pltpu.VMEM((k, _CH), jnp.int32),
                       pltpu.VMEM((_KF * _CH, w), jnp.float32),
                       pltpu.SemaphoreType.DMA],
    )
    def kern(table, idx, out, idx_v, buf, sem):
        c = lax.axis_index("c")
        s = lax.axis_index("s")
        wid = s * 2 + c
        pltpu.sync_copy(idx.at[pl.ds(wid * k, k)], idx_v)

        def grp(g, _):
            descs = [
                pltpu.async_copy(table.at[idx_v.at[g * _KF + j]],
                                 buf.at[pl.ds(j * _CH, _CH)], sem)
                for j in range(_KF)
            ]
            for dsc in descs:
                dsc.wait()
            pltpu.sync_copy(
                buf, out.at[pl.ds(wid * rw + g * _KF * _CH, _KF * _CH)])
            return 0

        lax.fori_loop(0, ng, grp, 0)

    return kern


def _gather2(table, idx_s, idx_d):
    w = table.shape[1]
    return _mk_gather2(w, table.shape[0])(
        table, idx_s.reshape(-1, _CH), idx_d.reshape(-1, _CH))


def _gather1(table, idx):
    w = table.shape[1]
    return _mk_gather1(w, table.shape[0])(table, idx.reshape(-1, _CH))


def _scatter2(vals_s, idx_s, vals_d, idx_d):
    w = vals_s.shape[1]
    p0 = jax.ops.segment_sum(vals_s, idx_s, num_segments=ACC_N)
    p1 = jax.ops.segment_sum(vals_d, idx_d, num_segments=ACC_N)
    return jnp.stack([p0, p1])


def _scatter1(vals, idx):
    w = vals.shape[1]
    p0 = jax.ops.segment_sum(vals, idx, num_segments=ACC_N)
    return jnp.stack([p0, jnp.zeros_like(p0)])


def kernel(pos, atom_features, edge_index, solvent_index, emb_solv, gamma_emb,
           Wm1, bm1, Wn1, bn1, Wm2, bm2, Wn2, bn2, Wm3, bm3, Wn3, bn3):
    src = edge_index[0]
    dst = edge_index[1]
    pad_g = jnp.zeros((EPAD - E,), jnp.int32)
    pad_s = jnp.full((EPAD - E,), DUMMY, jnp.int32)
    src_g = jnp.concatenate([src, pad_g])
    dst_g = jnp.concatenate([dst, pad_g])
    src_s = jnp.concatenate([src, pad_s])
    dst_s = jnp.concatenate([dst, pad_s])

    token = emb_solv[solvent_index[0]][None, :]
    gamma = gamma_emb[solvent_index[0], 0].reshape(1, 1)
    t1 = jnp.concatenate([pos, atom_features, jnp.zeros((N, 10), jnp.float32)],
                         axis=1)

    b1 = bm1[None, :]
    b2 = bm2[None, :]
    b3 = bm3[None, :]

    t1s, t1d = _gather2(t1, src_g, dst_g)
    m1, a1, geo = _f1_call(t1s, t1d, Wm1[0:3], Wm1[3:6], Wm1[6:38], b1)
    agg1 = _scatter1(m1, dst_s)
    h1, hp1 = _n1_call(agg1, t1, Wn1[0:64], Wn1[64:67], Wn1[67:131],
                       bn1[None, :], token)

    h1s, h1d = _gather2(h1, src_g, dst_g)
    m2, a2 = _f23_call(h1s, h1d, geo, Wm2[0:64], Wm2[64:128], Wm2[128:160], b2)
    agg2 = _scatter1(m2, dst_s)
    h2, hp2 = _n2_call(agg2, h1, Wn2[0:64], Wn2[64:128], Wn2[128:192],
                       bn2[None, :], token)

    h2s, h2d = _gather2(h2, src_g, dst_g)
    m3, a3 = _f23_call(h2s, h2d, geo, Wm3[0:64], Wm3[64:128], Wm3[128:160], b3)
    agg3 = _scatter1(m3, dst_s)
    tb, es_sum = _n3gb_call(agg3, h2, t1, Wn3[0:64], Wn3[64:128], Wn3[128:192],
                            bn3[None, :], token, gamma)

    tbs, tbd = _gather2(tb, src_g, dst_g)
    ep_sum, gdgb, vsbs, vsbd = _gbedge_call(tbs, tbd, geo)
    sb = _scatter2(vsbs, src_s, vsbd, dst_s)
    gagg3, gxdir3 = _gbnode_call(tb, sb, Wn3[0:64].T, Wn3[64:128].T, gamma)

    gg3 = _gather1(gagg3, dst_g)
    vs3, vd3, gek3 = _bedge3_call(gg3, a3, Wm3[0:64].T, Wm3[64:128].T,
                                  Wm3[128:160].T)
    sx3 = _scatter2(vs3, src_s, vd3, dst_s)
    gagg2, gxdir2 = _bnode_call(gxdir3, sx3, hp2, Wn2[0:64].T, Wn2[64:128].T)

    gg2 = _gather1(gagg2, dst_g)
    vs2, vd2, gek23 = _bedge2_call(gg2, a2, gek3, Wm2[0:64].T, Wm2[64:128].T,
                                   Wm2[128:160].T)
    sx2 = _scatter2(vs2, src_s, vd2, dst_s)
    gagg1 = _b1node_call(gxdir2, sx2, hp1, Wn1[0:64].T)

    gg1 = _gather1(gagg1, dst_g)
    fs, fd = _bfinal_call(gg1, a1, gek23, geo, gdgb, Wm1[6:38].T)
    sf = _scatter2(fs, src_s, fd, dst_s)
    fpad = _fadd_call(sf)

    forces = fpad[:, 0:3]
    energy = (ep_sum + es_sum).reshape(1, 1)
    return energy, forces


# speedup vs baseline: 3.1273x; 1.5967x over previous
import functools

import jax
import jax.numpy as jnp
import numpy as np
from jax import lax
from jax.experimental import pallas as pl
from jax.experimental.pallas import tpu as pltpu
from jax.experimental.pallas import tpu_sc as plsc

N = 10000
E = 160000
H = 64
NK = 32
RADIUS = 0.6
FRACTION = 0.1
SCALING = 2.0
EPS_SOLVENT = 78.5

BLKE = 1280
EPAD = 163840
GRID_E = EPAD // BLKE
BLKN = 2000
GRID_N = N // BLKN
ACC_N = 10240
DUMMY = N

_M = 2.0 * (RADIUS - 0.1) / (NK + 1)
_C0 = 0.1 + _M
_CSTEP = ((RADIUS - _M) - (0.1 + _M)) / (NK - 1)
_FOURPI = float(4.0 * np.pi)


def _centers():
    ci = lax.broadcasted_iota(jnp.int32, (1, NK), 1).astype(jnp.float32)
    return _C0 + ci * _CSTEP


def _sig(x):
    return 1.0 / (1.0 + jnp.exp(-x))


def _silu(x):
    return x * _sig(x)


def _dsilu(x):
    s = _sig(x)
    return s * (1.0 + x * (1.0 - s))


def _ek_from_d(d):
    k = d - _centers()
    t = 1.0 - (k / _M) ** 2
    tm = jnp.maximum(t, 0.0)
    return tm * tm * tm


def _dek_dd(d):
    k = d - _centers()
    t = 1.0 - (k / _M) ** 2
    tm = jnp.maximum(t, 0.0)
    return 3.0 * tm * tm * (-2.0 * k / (_M * _M))


def _wfull(shape):
    return pl.BlockSpec(shape, lambda i: tuple(0 for _ in shape))


_VE = lambda i: (i, 0)
_VN = lambda i: (i, 0)


def _part_specs(w):
    return [pl.BlockSpec((1, BLKN, w), lambda i: (0, i, 0)),
            pl.BlockSpec((1, BLKN, w), lambda i: (1, i, 0))]


def _f1_body(t1s, t1d, wms, wmd, wmk, bm, m_ref, a_ref, geo_ref):
    s_ = t1s[...]
    d_ = t1d[...]
    diff = s_[:, 0:3] - d_[:, 0:3]
    d2 = jnp.sum(diff * diff, axis=1, keepdims=True) + 1e-12
    dd = jnp.sqrt(d2)
    ek = _ek_from_d(dd)
    a = (jnp.dot(s_[:, 3:6], wms[...]) + jnp.dot(d_[:, 3:6], wmd[...])
         + jnp.dot(ek, wmk[...]) + bm[...])
    a_ref[...] = a
    m_ref[...] = _silu(a)
    geo_ref[...] = jnp.concatenate(
        [dd, diff, jnp.zeros((BLKE, 4), jnp.float32)], axis=1)


def _f23_body(hs, hd, geo, wms, wmd, wmk, bm, m_ref, a_ref):
    ek = _ek_from_d(geo[...][:, 0:1])
    a = (jnp.dot(hs[...], wms[...]) + jnp.dot(hd[...], wmd[...])
         + jnp.dot(ek, wmk[...]) + bm[...])
    a_ref[...] = a
    m_ref[...] = _silu(a)


def _n1_body(aggA, aggB, t1, wn_agg, wn_x, wn_tok, bn, token, h_ref, hp_ref):
    agg = aggA[...][0] + aggB[...][0]
    tokc = jnp.dot(token[...], wn_tok[...]) + bn[...]
    hp = jnp.dot(agg, wn_agg[...]) + jnp.dot(t1[...][:, 3:6], wn_x[...]) + tokc
    hp_ref[...] = hp
    h_ref[...] = _silu(hp)


def _n2_body(aggA, aggB, x, wn_agg, wn_x, wn_tok, bn, token, h_ref, hp_ref):
    agg = aggA[...][0] + aggB[...][0]
    tokc = jnp.dot(token[...], wn_tok[...]) + bn[...]
    hp = jnp.dot(agg, wn_agg[...]) + jnp.dot(x[...], wn_x[...]) + tokc
    hp_ref[...] = hp
    h_ref[...] = _silu(hp)


def _n3gb_body(aggA, aggB, h2, t1, wn_agg, wn_x, wn_tok, bn, token, gamma,
               tb_ref, es_ref):
    agg = aggA[...][0] + aggB[...][0]
    tokc = jnp.dot(token[...], wn_tok[...]) + bn[...]
    c = jnp.dot(agg, wn_agg[...]) + jnp.dot(h2[...], wn_x[...]) + tokc
    q = t1[...][:, 3:4]
    sc0 = _sig(c[:, 0:1])
    sc1 = _sig(c[:, 1:2])
    B = 0.1 + 0.4 * sc1
    sa = FRACTION * sc0
    e_self = -0.5 * q * q / B * (1.0 - 1.0 / EPS_SOLVENT)
    e_sa = gamma[...][0, 0] * sa * _FOURPI * (B + 0.14) ** 2
    part = jnp.sum(e_self + e_sa)

    @pl.when(pl.program_id(0) == 0)
    def _():
        es_ref[...] = jnp.zeros_like(es_ref)

    es_ref[...] += jnp.reshape(part, (1, 1))
    tb_ref[...] = jnp.concatenate(
        [B, q, sc0, sc1, jnp.zeros((BLKN, 12), jnp.float32)], axis=1)


def _gbedge_body(tbs, tbd, geo, ep_ref, gdgb_ref, vsbs_ref, vsbd_ref):
    s_ = tbs[...]
    d_ = tbd[...]
    Bs = s_[:, 0:1]
    qs = s_[:, 1:2]
    Bd = d_[:, 0:1]
    qd = d_[:, 1:2]
    dd = geo[...][:, 0:1]
    d2 = dd * dd
    Bij = Bs * Bd
    u = jnp.exp(-d2 / (4.0 * Bij))
    fgb2 = d2 + Bij * u
    fgb = jnp.sqrt(fgb2)
    qq = qs * qd
    epair = -0.5 * qq / fgb
    rid = pl.program_id(0) * BLKE + lax.broadcasted_iota(jnp.int32, (BLKE, 1), 0)
    part = jnp.sum(jnp.where(rid < E, epair, 0.0))

    @pl.when(pl.program_id(0) == 0)
    def _():
        ep_ref[...] = jnp.zeros_like(ep_ref)

    ep_ref[...] += jnp.reshape(part, (1, 1))
    gfgb = 0.5 * qq / fgb2
    gdgb_ref[...] = gfgb * dd * (1.0 - 0.25 * u) / fgb
    gBij = gfgb * u * (1.0 + d2 / (4.0 * Bij)) / (2.0 * fgb)
    z = jnp.zeros((BLKE, 15), jnp.float32)
    vsbs_ref[...] = jnp.concatenate([gBij * Bd, z], axis=1)
    vsbd_ref[...] = jnp.concatenate([gBij * Bs, z], axis=1)


def _gbnode_body(tb, sbA, sbB, wn3_aggT, wn3_xT, gamma, gagg_ref, gxdir_ref):
    t_ = tb[...]
    B = t_[:, 0:1]
    q = t_[:, 1:2]
    sc0 = t_[:, 2:3]
    sc1 = t_[:, 3:4]
    g = gamma[...][0, 0]
    gB = (sbA[...][0][:, 0:1] + sbB[...][0][:, 0:1]
          + 0.5 * q * q / (B * B) * (1.0 - 1.0 / EPS_SOLVENT)
          + g * (FRACTION * sc0) * 2.0 * _FOURPI * (B + 0.14))
    gsa = g * _FOURPI * (B + 0.14) ** 2
    gc1 = gB * 0.4 * sc1 * (1.0 - sc1)
    gc0 = gsa * FRACTION * sc0 * (1.0 - sc0)
    Gc = jnp.concatenate([gc0, gc1], axis=1)
    gagg_ref[...] = jnp.dot(Gc, wn3_aggT[...])
    gxdir_ref[...] = jnp.dot(Gc, wn3_xT[...])


def _bedge3_body(gg, a, wmsT, wmdT, wmkT, vs_ref, vd_ref, gek_ref):
    Ga = gg[...] * _dsilu(a[...])
    vs_ref[...] = jnp.dot(Ga, wmsT[...])
    vd_ref[...] = jnp.dot(Ga, wmdT[...])
    gek_ref[...] = jnp.dot(Ga, wmkT[...])


def _bedge2_body(gg, a, gek_in, wmsT, wmdT, wmkT, vs_ref, vd_ref, gek_ref):
    Ga = gg[...] * _dsilu(a[...])
    vs_ref[...] = jnp.dot(Ga, wmsT[...])
    vd_ref[...] = jnp.dot(Ga, wmdT[...])
    gek_ref[...] = gek_in[...] + jnp.dot(Ga, wmkT[...])


def _bnode_body(gxdir, sxA, sxB, hp, wn_aggT, wn_xT, gagg_ref, gxn_ref):
    Gh = (gxdir[...] + sxA[...][0] + sxB[...][0]) * _dsilu(hp[...])
    gagg_ref[...] = jnp.dot(Gh, wn_aggT[...])
    gxn_ref[...] = jnp.dot(Gh, wn_xT[...])


def _b1node_body(gxdir, sxA, sxB, hp, wn_aggT, gagg_ref):
    Gh = (gxdir[...] + sxA[...][0] + sxB[...][0]) * _dsilu(hp[...])
    gagg_ref[...] = jnp.dot(Gh, wn_aggT[...])


def _bfinal_body(gg1, a1, gek23, geo, gdgb, wm1kT, fs_ref, fd_ref):
    Ga1 = gg1[...] * _dsilu(a1[...])
    gek = gek23[...] + jnp.dot(Ga1, wm1kT[...])
    g_ = geo[...]
    dd = g_[:, 0:1]
    diff = g_[:, 1:4]
    gd = gdgb[...] + jnp.sum(gek * _dek_dd(dd), axis=1, keepdims=True)
    f = (-gd / dd) * diff
    z = jnp.zeros((BLKE, 13), jnp.float32)
    fs_ref[...] = jnp.concatenate([f, z], axis=1)
    fd_ref[...] = jnp.concatenate([-f, z], axis=1)


def _fadd_body(pA, pB, out_ref):
    out_ref[...] = pA[...][0] + pB[...][0]


def _edge_spec(w):
    return pl.BlockSpec((BLKE, w), _VE)


def _node_spec(w):
    return pl.BlockSpec((BLKN, w), _VN)


def _eshape(w):
    return jax.ShapeDtypeStruct((EPAD, w), jnp.float32)


def _nshape(w):
    return jax.ShapeDtypeStruct((N, w), jnp.float32)


_SCALAR_SPEC = pl.BlockSpec((1, 1), lambda i: (0, 0))
_SCALAR_SHAPE = jax.ShapeDtypeStruct((1, 1), jnp.float32)


def _f1_call(t1s, t1d, wms, wmd, wmk, bm):
    return pl.pallas_call(
        _f1_body,
        grid=(GRID_E,),
        in_specs=[_edge_spec(16), _edge_spec(16), _wfull((3, H)), _wfull((3, H)),
                  _wfull((NK, H)), _wfull((1, H))],
        out_specs=[_edge_spec(H), _edge_spec(H), _edge_spec(8)],
        out_shape=[_eshape(H), _eshape(H), _eshape(8)],
    )(t1s, t1d, wms, wmd, wmk, bm)


def _f23_call(hs, hd, geo, wms, wmd, wmk, bm):
    return pl.pallas_call(
        _f23_body,
        grid=(GRID_E,),
        in_specs=[_edge_spec(H), _edge_spec(H), _edge_spec(8), _wfull((H, H)),
                  _wfull((H, H)), _wfull((NK, H)), _wfull((1, H))],
        out_specs=[_edge_spec(H), _edge_spec(H)],
        out_shape=[_eshape(H), _eshape(H)],
    )(hs, hd, geo, wms, wmd, wmk, bm)


def _n1_call(agg, t1, wn_agg, wn_x, wn_tok, bn, token):
    sA, sB = _part_specs(H)
    return pl.pallas_call(
        _n1_body,
        grid=(GRID_N,),
        in_specs=[sA, sB, _node_spec(16), _wfull((H, H)), _wfull((3, H)),
                  _wfull((H, H)), _wfull((1, H)), _wfull((1, H))],
        out_specs=[_node_spec(H), _node_spec(H)],
        out_shape=[_nshape(H), _nshape(H)],
    )(agg, agg, t1, wn_agg, wn_x, wn_tok, bn, token)


def _n2_call(agg, x, wn_agg, wn_x, wn_tok, bn, token):
    sA, sB = _part_specs(H)
    return pl.pallas_call(
        _n2_body,
        grid=(GRID_N,),
        in_specs=[sA, sB, _node_spec(H), _wfull((H, H)), _wfull((H, H)),
                  _wfull((H, H)), _wfull((1, H)), _wfull((1, H))],
        out_specs=[_node_spec(H), _node_spec(H)],
        out_shape=[_nshape(H), _nshape(H)],
    )(agg, agg, x, wn_agg, wn_x, wn_tok, bn, token)


def _n3gb_call(agg, h2, t1, wn_agg, wn_x, wn_tok, bn, token, gamma):
    sA, sB = _part_specs(H)
    return pl.pallas_call(
        _n3gb_body,
        grid=(GRID_N,),
        in_specs=[sA, sB, _node_spec(H), _node_spec(16), _wfull((H, 2)),
                  _wfull((H, 2)), _wfull((H, 2)), _wfull((1, 2)),
                  _wfull((1, H)), _wfull((1, 1))],
        out_specs=[_node_spec(16), _SCALAR_SPEC],
        out_shape=[_nshape(16), _SCALAR_SHAPE],
    )(agg, agg, h2, t1, wn_agg, wn_x, wn_tok, bn, token, gamma)


def _gbedge_call(tbs, tbd, geo):
    return pl.pallas_call(
        _gbedge_body,
        grid=(GRID_E,),
        in_specs=[_edge_spec(16), _edge_spec(16), _edge_spec(8)],
        out_specs=[_SCALAR_SPEC, _edge_spec(1), _edge_spec(16), _edge_spec(16)],
        out_shape=[_SCALAR_SHAPE, _eshape(1), _eshape(16), _eshape(16)],
    )(tbs, tbd, geo)


def _gbnode_call(tb, sb, wn3_aggT, wn3_xT, gamma):
    sA, sB = _part_specs(16)
    return pl.pallas_call(
        _gbnode_body,
        grid=(GRID_N,),
        in_specs=[_node_spec(16), sA, sB, _wfull((2, H)), _wfull((2, H)),
                  _wfull((1, 1))],
        out_specs=[_node_spec(H), _node_spec(H)],
        out_shape=[_nshape(H), _nshape(H)],
    )(tb, sb, sb, wn3_aggT, wn3_xT, gamma)


def _bedge3_call(gg, a, wmsT, wmdT, wmkT):
    return pl.pallas_call(
        _bedge3_body,
        grid=(GRID_E,),
        in_specs=[_edge_spec(H), _edge_spec(H), _wfull((H, H)), _wfull((H, H)),
                  _wfull((H, NK))],
        out_specs=[_edge_spec(H), _edge_spec(H), _edge_spec(NK)],
        out_shape=[_eshape(H), _eshape(H), _eshape(NK)],
    )(gg, a, wmsT, wmdT, wmkT)


def _bedge2_call(gg, a, gek_in, wmsT, wmdT, wmkT):
    return pl.pallas_call(
        _bedge2_body,
        grid=(GRID_E,),
        in_specs=[_edge_spec(H), _edge_spec(H), _edge_spec(NK), _wfull((H, H)),
                  _wfull((H, H)), _wfull((H, NK))],
        out_specs=[_edge_spec(H), _edge_spec(H), _edge_spec(NK)],
        out_shape=[_eshape(H), _eshape(H), _eshape(NK)],
    )(gg, a, gek_in, wmsT, wmdT, wmkT)


def _bnode_call(gxdir, sx, hp, wn_aggT, wn_xT):
    sA, sB = _part_specs(H)
    return pl.pallas_call(
        _bnode_body,
        grid=(GRID_N,),
        in_specs=[_node_spec(H), sA, sB, _node_spec(H), _wfull((H, H)),
                  _wfull((H, H))],
        out_specs=[_node_spec(H), _node_spec(H)],
        out_shape=[_nshape(H), _nshape(H)],
    )(gxdir, sx, sx, hp, wn_aggT, wn_xT)


def _b1node_call(gxdir, sx, hp, wn_aggT):
    sA, sB = _part_specs(H)
    return pl.pallas_call(
        _b1node_body,
        grid=(GRID_N,),
        in_specs=[_node_spec(H), sA, sB, _node_spec(H), _wfull((H, H))],
        out_specs=_node_spec(H),
        out_shape=_nshape(H),
    )(gxdir, sx, sx, hp, wn_aggT)


def _bfinal_call(gg1, a1, gek23, geo, gdgb, wm1kT):
    return pl.pallas_call(
        _bfinal_body,
        grid=(GRID_E,),
        in_specs=[_edge_spec(H), _edge_spec(H), _edge_spec(NK), _edge_spec(8),
                  _edge_spec(1), _wfull((H, NK))],
        out_specs=[_edge_spec(16), _edge_spec(16)],
        out_shape=[_eshape(16), _eshape(16)],
    )(gg1, a1, gek23, geo, gdgb, wm1kT)


def _fadd_call(sf):
    sA, sB = _part_specs(16)
    return pl.pallas_call(
        _fadd_body,
        grid=(GRID_N,),
        in_specs=[sA, sB],
        out_specs=_node_spec(16),
        out_shape=_nshape(16),
    )(sf, sf)


_KF = 8
_CH = 128
_NW = 32
_RPT = ACC_N // 16


def _sc_mesh():
    return plsc.VectorSubcoreMesh(core_axis_name="c", subcore_axis_name="s")


_SC_PARAMS = pltpu.CompilerParams(use_tc_tiling_on_sc=False)


@functools.lru_cache(maxsize=None)
def _mk_gather2(w, nt):
    rw = EPAD // 16
    k = rw // _CH
    ng = k // _KF

    @functools.partial(
        pl.kernel,
        out_type=[jax.ShapeDtypeStruct((EPAD, w), jnp.float32),
                  jax.ShapeDtypeStruct((EPAD, w), jnp.float32)],
        mesh=_sc_mesh(),
        compiler_params=_SC_PARAMS,
        scratch_types=[pltpu.VMEM((k, _CH), jnp.int32),
                       pltpu.VMEM((_KF * _CH, w), jnp.float32),
                       pltpu.SemaphoreType.DMA],
    )
    def kern(table, idxs, idxd, outs, outd, idx_v, buf, sem):
        c = lax.axis_index("c")
        s = lax.axis_index("s")
        wid = s * 2 + c
        lw = wid % 16

        def process(idx_hbm, out_hbm):
            pltpu.sync_copy(idx_hbm.at[pl.ds(lw * k, k)], idx_v)

            def grp(g, _):
                descs = [
                    pltpu.async_copy(table.at[idx_v.at[g * _KF + j]],
                                     buf.at[pl.ds(j * _CH, _CH)], sem)
                    for j in range(_KF)
                ]
                for dsc in descs:
                    dsc.wait()
                pltpu.sync_copy(
                    buf, out_hbm.at[pl.ds(lw * rw + g * _KF * _CH, _KF * _CH)])
                return 0

            lax.fori_loop(0, ng, grp, 0)

        @pl.when(wid < 16)
        def _():
            process(idxs, outs)

        @pl.when(wid >= 16)
        def _():
            process(idxd, outd)

    return kern


@functools.lru_cache(maxsize=None)
def _mk_gather1(w, nt):
    rw = EPAD // _NW
    k = rw // _CH
    ng = k // _KF

    @functools.partial(
        pl.kernel,
        out_type=jax.ShapeDtypeStruct((EPAD, w), jnp.float32),
        mesh=_sc_mesh(),
        compiler_params=_SC_PARAMS,
        scratch_types=[pltpu.VMEM((k, _CH), jnp.int32),
                       pltpu.VMEM((_KF * _CH, w), jnp.float32),
                       pltpu.SemaphoreType.DMA],
    )
    def kern(table, idx, out, idx_v, buf, sem):
        c = lax.axis_index("c")
        s = lax.axis_index("s")
        wid = s * 2 + c
        pltpu.sync_copy(idx.at[pl.ds(wid * k, k)], idx_v)

        def grp(g, _):
            descs = [
                pltpu.async_copy(table.at[idx_v.at[g * _KF + j]],
                                 buf.at[pl.ds(j * _CH, _CH)], sem)
                for j in range(_KF)
            ]
            for dsc in descs:
                dsc.wait()
            pltpu.sync_copy(
                buf, out.at[pl.ds(wid * rw + g * _KF * _CH, _KF * _CH)])
            return 0

        lax.fori_loop(0, ng, grp, 0)

    return kern


def _gather2(table, idx_s, idx_d):
    w = table.shape[1]
    return _mk_gather2(w, table.shape[0])(
        table, idx_s.reshape(-1, _CH), idx_d.reshape(-1, _CH))


def _gather1(table, idx):
    w = table.shape[1]
    return _mk_gather1(w, table.shape[0])(table, idx.reshape(-1, _CH))


@functools.lru_cache(maxsize=None)
def _mk_scatter2(w):
    rw = EPAD // 16
    k = rw // _CH
    ng = k // _KF

    @functools.partial(
        pl.kernel,
        out_type=jax.ShapeDtypeStruct((2, ACC_N, w), jnp.float32),
        mesh=_sc_mesh(),
        compiler_params=_SC_PARAMS,
        scratch_types=[pltpu.VMEM((k, _CH), jnp.int32),
                       pltpu.VMEM((_KF * _CH, w), jnp.float32),
                       pltpu.VMEM_SHARED((ACC_N, w), jnp.float32)],
    )
    def kern(vals_s, vals_d, idxs, idxd, zeros, out, idx_v, buf, acc):
        c = lax.axis_index("c")
        s = lax.axis_index("s")
        wid = s * 2 + c
        lw = wid % 16
        pltpu.sync_copy(zeros.at[pl.ds(s * _RPT, _RPT)],
                        acc.at[pl.ds(s * _RPT, _RPT)])
        plsc.subcore_barrier()

        def process(vals_hbm, idx_hbm):
            pltpu.sync_copy(idx_hbm.at[pl.ds(lw * k, k)], idx_v)

            def grp(g, _):
                pltpu.sync_copy(
                    vals_hbm.at[pl.ds(lw * rw + g * _KF * _CH, _KF * _CH)], buf)
                for j in range(_KF):
                    pltpu.sync_copy(buf.at[pl.ds(j * _CH, _CH)],
                                    acc.at[idx_v.at[g * _KF + j]], add=True)
                return 0

            lax.fori_loop(0, ng, grp, 0)

        @pl.when(wid < 16)
        def _():
            process(vals_s, idxs)

        @pl.when(wid >= 16)
        def _():
            process(vals_d, idxd)

        plsc.subcore_barrier()
        pltpu.sync_copy(acc.at[pl.ds(s * _RPT, _RPT)],
                        out.at[c, pl.ds(s * _RPT, _RPT)])

    return kern


@functools.lru_cache(maxsize=None)
def _mk_scatter1(w):
    rw = EPAD // _NW
    k = rw // _CH
    ng = k // _KF

    @functools.partial(
        pl.kernel,
        out_type=jax.ShapeDtypeStruct((2, ACC_N, w), jnp.float32),
        mesh=_sc_mesh(),
        compiler_params=_SC_PARAMS,
        scratch_types=[pltpu.VMEM((k, _CH), jnp.int32),
                       pltpu.VMEM((_KF * _CH, w), jnp.float32),
                       pltpu.VMEM_SHARED((ACC_N, w), jnp.float32)],
    )
    def kern(vals, idx, zeros, out, idx_v, buf, acc):
        c = lax.axis_index("c")
        s = lax.axis_index("s")
        wid = s * 2 + c
        pltpu.sync_copy(zeros.at[pl.ds(s * _RPT, _RPT)],
                        acc.at[pl.ds(s * _RPT, _RPT)])
        plsc.subcore_barrier()
        pltpu.sync_copy(idx.at[pl.ds(wid * k, k)], idx_v)

        def grp(g, _):
            pltpu.sync_copy(vals.at[pl.ds(wid * rw + g * _KF * _CH, _KF * _CH)],
                            buf)
            for j in range(_KF):
                pltpu.sync_copy(buf.at[pl.ds(j * _CH, _CH)],
                                acc.at[idx_v.at[g * _KF + j]], add=True)
            return 0

        lax.fori_loop(0, ng, grp, 0)
        plsc.subcore_barrier()
        pltpu.sync_copy(acc.at[pl.ds(s * _RPT, _RPT)],
                        out.at[c, pl.ds(s * _RPT, _RPT)])

    return kern


def _scatter2(vals_s, idx_s, vals_d, idx_d):
    w = vals_s.shape[1]
    zeros = jnp.zeros((ACC_N, w), jnp.float32)
    return _mk_scatter2(w)(vals_s, vals_d, idx_s.reshape(-1, _CH),
                           idx_d.reshape(-1, _CH), zeros)


def _scatter1(vals, idx):
    w = vals.shape[1]
    zeros = jnp.zeros((ACC_N, w), jnp.float32)
    return _mk_scatter1(w)(vals, idx.reshape(-1, _CH), zeros)


def kernel(pos, atom_features, edge_index, solvent_index, emb_solv, gamma_emb,
           Wm1, bm1, Wn1, bn1, Wm2, bm2, Wn2, bn2, Wm3, bm3, Wn3, bn3):
    src = edge_index[0]
    dst = edge_index[1]
    pad_g = jnp.zeros((EPAD - E,), jnp.int32)
    pad_s = jnp.full((EPAD - E,), DUMMY, jnp.int32)
    src_g = jnp.concatenate([src, pad_g])
    dst_g = jnp.concatenate([dst, pad_g])
    src_s = jnp.concatenate([src, pad_s])
    dst_s = jnp.concatenate([dst, pad_s])

    token = emb_solv[solvent_index[0]][None, :]
    gamma = gamma_emb[solvent_index[0], 0].reshape(1, 1)
    t1 = jnp.concatenate([pos, atom_features, jnp.zeros((N, 10), jnp.float32)],
                         axis=1)

    b1 = bm1[None, :]
    b2 = bm2[None, :]
    b3 = bm3[None, :]

    t1s, t1d = _gather2(t1, src_g, dst_g)
    m1, a1, geo = _f1_call(t1s, t1d, Wm1[0:3], Wm1[3:6], Wm1[6:38], b1)
    agg1 = _scatter1(m1, dst_s)
    h1, hp1 = _n1_call(agg1, t1, Wn1[0:64], Wn1[64:67], Wn1[67:131],
                       bn1[None, :], token)

    h1s, h1d = _gather2(h1, src_g, dst_g)
    m2, a2 = _f23_call(h1s, h1d, geo, Wm2[0:64], Wm2[64:128], Wm2[128:160], b2)
    agg2 = _scatter1(m2, dst_s)
    h2, hp2 = _n2_call(agg2, h1, Wn2[0:64], Wn2[64:128], Wn2[128:192],
                       bn2[None, :], token)

    h2s, h2d = _gather2(h2, src_g, dst_g)
    m3, a3 = _f23_call(h2s, h2d, geo, Wm3[0:64], Wm3[64:128], Wm3[128:160], b3)
    agg3 = _scatter1(m3, dst_s)
    tb, es_sum = _n3gb_call(agg3, h2, t1, Wn3[0:64], Wn3[64:128], Wn3[128:192],
                            bn3[None, :], token, gamma)

    tbs, tbd = _gather2(tb, src_g, dst_g)
    ep_sum, gdgb, vsbs, vsbd = _gbedge_call(tbs, tbd, geo)
    sb = _scatter2(vsbs, src_s, vsbd, dst_s)
    gagg3, gxdir3 = _gbnode_call(tb, sb, Wn3[0:64].T, Wn3[64:128].T, gamma)

    gg3 = _gather1(gagg3, dst_g)
    vs3, vd3, gek3 = _bedge3_call(gg3, a3, Wm3[0:64].T, Wm3[64:128].T,
                                  Wm3[128:160].T)
    sx3 = _scatter2(vs3, src_s, vd3, dst_s)
    gagg2, gxdir2 = _bnode_call(gxdir3, sx3, hp2, Wn2[0:64].T, Wn2[64:128].T)

    gg2 = _gather1(gagg2, dst_g)
    vs2, vd2, gek23 = _bedge2_call(gg2, a2, gek3, Wm2[0:64].T, Wm2[64:128].T,
                                   Wm2[128:160].T)
    sx2 = _scatter2(vs2, src_s, vd2, dst_s)
    gagg1 = _b1node_call(gxdir2, sx2, hp1, Wn1[0:64].T)

    gg1 = _gather1(gagg1, dst_g)
    fs, fd = _bfinal_call(gg1, a1, gek23, geo, gdgb, Wm1[6:38].T)
    sf = _scatter2(fs, src_s, fd, dst_s)
    fpad = _fadd_call(sf)

    forces = fpad[:, 0:3]
    energy = (ep_sum + es_sum).reshape(1, 1)
    return energy, forces


# speedup vs baseline: 3.1789x; 1.0165x over previous
import functools

import jax
import jax.numpy as jnp
import numpy as np
from jax import lax
from jax.experimental import pallas as pl
from jax.experimental.pallas import tpu as pltpu
from jax.experimental.pallas import tpu_sc as plsc

N = 10000
E = 160000
H = 64
NK = 32
RADIUS = 0.6
FRACTION = 0.1
SCALING = 2.0
EPS_SOLVENT = 78.5

BLKE = 1280
EPAD = 163840
GRID_E = EPAD // BLKE
BLKN = 2000
GRID_N = N // BLKN
ACC_N = 10240
DUMMY = N

_M = 2.0 * (RADIUS - 0.1) / (NK + 1)
_C0 = 0.1 + _M
_CSTEP = ((RADIUS - _M) - (0.1 + _M)) / (NK - 1)
_FOURPI = float(4.0 * np.pi)


def _centers():
    ci = lax.broadcasted_iota(jnp.int32, (1, NK), 1).astype(jnp.float32)
    return _C0 + ci * _CSTEP


def _sig(x):
    return 1.0 / (1.0 + jnp.exp(-x))


def _silu(x):
    return x * _sig(x)


def _dsilu(x):
    s = _sig(x)
    return s * (1.0 + x * (1.0 - s))


def _ek_from_d(d):
    k = d - _centers()
    t = 1.0 - (k / _M) ** 2
    tm = jnp.maximum(t, 0.0)
    return tm * tm * tm


def _dek_dd(d):
    k = d - _centers()
    t = 1.0 - (k / _M) ** 2
    tm = jnp.maximum(t, 0.0)
    return 3.0 * tm * tm * (-2.0 * k / (_M * _M))


def _wfull(shape):
    return pl.BlockSpec(shape, lambda i: tuple(0 for _ in shape))


_VE = lambda i: (i, 0)
_VN = lambda i: (i, 0)


def _part_specs(w):
    return [pl.BlockSpec((1, BLKN, w), lambda i: (0, i, 0)),
            pl.BlockSpec((1, BLKN, w), lambda i: (1, i, 0))]


def _f1_body(t1s, t1d, wms, wmd, wmk, bm, m_ref, a_ref, geo_ref):
    s_ = t1s[...]
    d_ = t1d[...]
    diff = s_[:, 0:3] - d_[:, 0:3]
    d2 = jnp.sum(diff * diff, axis=1, keepdims=True) + 1e-12
    dd = jnp.sqrt(d2)
    ek = _ek_from_d(dd)
    a = (jnp.dot(s_[:, 3:6], wms[...]) + jnp.dot(d_[:, 3:6], wmd[...])
         + jnp.dot(ek, wmk[...]) + bm[...])
    a_ref[...] = a
    m_ref[...] = _silu(a)
    geo_ref[...] = jnp.concatenate(
        [dd, diff, jnp.zeros((BLKE, 4), jnp.float32)], axis=1)


def _f23_body(hs, hd, geo, wms, wmd, wmk, bm, m_ref, a_ref):
    ek = _ek_from_d(geo[...][:, 0:1])
    a = (jnp.dot(hs[...], wms[...]) + jnp.dot(hd[...], wmd[...])
         + jnp.dot(ek, wmk[...]) + bm[...])
    a_ref[...] = a
    m_ref[...] = _silu(a)


def _n1_body(aggA, aggB, t1, wn_agg, wn_x, wn_tok, bn, token, h_ref, hp_ref):
    agg = aggA[...][0] + aggB[...][0]
    tokc = jnp.dot(token[...], wn_tok[...]) + bn[...]
    hp = jnp.dot(agg, wn_agg[...]) + jnp.dot(t1[...][:, 3:6], wn_x[...]) + tokc
    hp_ref[...] = hp
    h_ref[...] = _silu(hp)


def _n2_body(aggA, aggB, x, wn_agg, wn_x, wn_tok, bn, token, h_ref, hp_ref):
    agg = aggA[...][0] + aggB[...][0]
    tokc = jnp.dot(token[...], wn_tok[...]) + bn[...]
    hp = jnp.dot(agg, wn_agg[...]) + jnp.dot(x[...], wn_x[...]) + tokc
    hp_ref[...] = hp
    h_ref[...] = _silu(hp)


def _n3gb_body(aggA, aggB, h2, t1, wn_agg, wn_x, wn_tok, bn, token, gamma,
               tb_ref, es_ref):
    agg = aggA[...][0] + aggB[...][0]
    tokc = jnp.dot(token[...], wn_tok[...]) + bn[...]
    c = jnp.dot(agg, wn_agg[...]) + jnp.dot(h2[...], wn_x[...]) + tokc
    q = t1[...][:, 3:4]
    sc0 = _sig(c[:, 0:1])
    sc1 = _sig(c[:, 1:2])
    B = 0.1 + 0.4 * sc1
    sa = FRACTION * sc0
    e_self = -0.5 * q * q / B * (1.0 - 1.0 / EPS_SOLVENT)
    e_sa = gamma[...][0, 0] * sa * _FOURPI * (B + 0.14) ** 2
    part = jnp.sum(e_self + e_sa)

    @pl.when(pl.program_id(0) == 0)
    def _():
        es_ref[...] = jnp.zeros_like(es_ref)

    es_ref[...] += jnp.reshape(part, (1, 1))
    tb_ref[...] = jnp.concatenate(
        [B, q, sc0, sc1, jnp.zeros((BLKN, 12), jnp.float32)], axis=1)


def _gbedge_body(tbs, tbd, geo, ep_ref, gdgb_ref, vsbs_ref, vsbd_ref):
    s_ = tbs[...]
    d_ = tbd[...]
    Bs = s_[:, 0:1]
    qs = s_[:, 1:2]
    Bd = d_[:, 0:1]
    qd = d_[:, 1:2]
    dd = geo[...][:, 0:1]
    d2 = dd * dd
    Bij = Bs * Bd
    u = jnp.exp(-d2 / (4.0 * Bij))
    fgb2 = d2 + Bij * u
    fgb = jnp.sqrt(fgb2)
    qq = qs * qd
    epair = -0.5 * qq / fgb
    rid = pl.program_id(0) * BLKE + lax.broadcasted_iota(jnp.int32, (BLKE, 1), 0)
    part = jnp.sum(jnp.where(rid < E, epair, 0.0))

    @pl.when(pl.program_id(0) == 0)
    def _():
        ep_ref[...] = jnp.zeros_like(ep_ref)

    ep_ref[...] += jnp.reshape(part, (1, 1))
    gfgb = 0.5 * qq / fgb2
    gdgb_ref[...] = gfgb * dd * (1.0 - 0.25 * u) / fgb
    gBij = gfgb * u * (1.0 + d2 / (4.0 * Bij)) / (2.0 * fgb)
    z = jnp.zeros((BLKE, 15), jnp.float32)
    vsbs_ref[...] = jnp.concatenate([gBij * Bd, z], axis=1)
    vsbd_ref[...] = jnp.concatenate([gBij * Bs, z], axis=1)


def _gbnode_body(tb, sbA, sbB, wn3_aggT, wn3_xT, gamma, gagg_ref, gxdir_ref):
    t_ = tb[...]
    B = t_[:, 0:1]
    q = t_[:, 1:2]
    sc0 = t_[:, 2:3]
    sc1 = t_[:, 3:4]
    g = gamma[...][0, 0]
    gB = (sbA[...][0][:, 0:1] + sbB[...][0][:, 0:1]
          + 0.5 * q * q / (B * B) * (1.0 - 1.0 / EPS_SOLVENT)
          + g * (FRACTION * sc0) * 2.0 * _FOURPI * (B + 0.14))
    gsa = g * _FOURPI * (B + 0.14) ** 2
    gc1 = gB * 0.4 * sc1 * (1.0 - sc1)
    gc0 = gsa * FRACTION * sc0 * (1.0 - sc0)
    Gc = jnp.concatenate([gc0, gc1], axis=1)
    gagg_ref[...] = jnp.dot(Gc, wn3_aggT[...])
    gxdir_ref[...] = jnp.dot(Gc, wn3_xT[...])


def _bedge3_body(gg, a, wmsT, wmdT, wmkT, vs_ref, vd_ref, gek_ref):
    Ga = gg[...] * _dsilu(a[...])
    vs_ref[...] = jnp.dot(Ga, wmsT[...])
    vd_ref[...] = jnp.dot(Ga, wmdT[...])
    gek_ref[...] = jnp.dot(Ga, wmkT[...])


def _bedge2_body(gg, a, gek_in, wmsT, wmdT, wmkT, vs_ref, vd_ref, gek_ref):
    Ga = gg[...] * _dsilu(a[...])
    vs_ref[...] = jnp.dot(Ga, wmsT[...])
    vd_ref[...] = jnp.dot(Ga, wmdT[...])
    gek_ref[...] = gek_in[...] + jnp.dot(Ga, wmkT[...])


def _bnode_body(gxdir, sxA, sxB, hp, wn_aggT, wn_xT, gagg_ref, gxn_ref):
    Gh = (gxdir[...] + sxA[...][0] + sxB[...][0]) * _dsilu(hp[...])
    gagg_ref[...] = jnp.dot(Gh, wn_aggT[...])
    gxn_ref[...] = jnp.dot(Gh, wn_xT[...])


def _b1node_body(gxdir, sxA, sxB, hp, wn_aggT, gagg_ref):
    Gh = (gxdir[...] + sxA[...][0] + sxB[...][0]) * _dsilu(hp[...])
    gagg_ref[...] = jnp.dot(Gh, wn_aggT[...])


def _bfinal_body(gg1, a1, gek23, geo, gdgb, wm1kT, fs_ref, fd_ref):
    Ga1 = gg1[...] * _dsilu(a1[...])
    gek = gek23[...] + jnp.dot(Ga1, wm1kT[...])
    g_ = geo[...]
    dd = g_[:, 0:1]
    diff = g_[:, 1:4]
    gd = gdgb[...] + jnp.sum(gek * _dek_dd(dd), axis=1, keepdims=True)
    f = (-gd / dd) * diff
    z = jnp.zeros((BLKE, 13), jnp.float32)
    fs_ref[...] = jnp.concatenate([f, z], axis=1)
    fd_ref[...] = jnp.concatenate([-f, z], axis=1)


def _fadd_body(pA, pB, out_ref):
    out_ref[...] = pA[...][0] + pB[...][0]


def _edge_spec(w):
    return pl.BlockSpec((BLKE, w), _VE)


def _node_spec(w):
    return pl.BlockSpec((BLKN, w), _VN)


def _eshape(w):
    return jax.ShapeDtypeStruct((EPAD, w), jnp.float32)


def _nshape(w):
    return jax.ShapeDtypeStruct((N, w), jnp.float32)


_SCALAR_SPEC = pl.BlockSpec((1, 1), lambda i: (0, 0))
_SCALAR_SHAPE = jax.ShapeDtypeStruct((1, 1), jnp.float32)


def _f1_call(t1s, t1d, wms, wmd, wmk, bm):
    return pl.pallas_call(
        _f1_body,
        grid=(GRID_E,),
        in_specs=[_edge_spec(16), _edge_spec(16), _wfull((3, H)), _wfull((3, H)),
                  _wfull((NK, H)), _wfull((1, H))],
        out_specs=[_edge_spec(H), _edge_spec(H), _edge_spec(8)],
        out_shape=[_eshape(H), _eshape(H), _eshape(8)],
    )(t1s, t1d, wms, wmd, wmk, bm)


def _f23_call(hs, hd, geo, wms, wmd, wmk, bm):
    return pl.pallas_call(
        _f23_body,
        grid=(GRID_E,),
        in_specs=[_edge_spec(H), _edge_spec(H), _edge_spec(8), _wfull((H, H)),
                  _wfull((H, H)), _wfull((NK, H)), _wfull((1, H))],
        out_specs=[_edge_spec(H), _edge_spec(H)],
        out_shape=[_eshape(H), _eshape(H)],
    )(hs, hd, geo, wms, wmd, wmk, bm)


def _n1_call(agg, t1, wn_agg, wn_x, wn_tok, bn, token):
    sA, sB = _part_specs(H)
    return pl.pallas_call(
        _n1_body,
        grid=(GRID_N,),
        in_specs=[sA, sB, _node_spec(16), _wfull((H, H)), _wfull((3, H)),
                  _wfull((H, H)), _wfull((1, H)), _wfull((1, H))],
        out_specs=[_node_spec(H), _node_spec(H)],
        out_shape=[_nshape(H), _nshape(H)],
    )(agg, agg, t1, wn_agg, wn_x, wn_tok, bn, token)


def _n2_call(agg, x, wn_agg, wn_x, wn_tok, bn, token):
    sA, sB = _part_specs(H)
    return pl.pallas_call(
        _n2_body,
        grid=(GRID_N,),
        in_specs=[sA, sB, _node_spec(H), _wfull((H, H)), _wfull((H, H)),
                  _wfull((H, H)), _wfull((1, H)), _wfull((1, H))],
        out_specs=[_node_spec(H), _node_spec(H)],
        out_shape=[_nshape(H), _nshape(H)],
    )(agg, agg, x, wn_agg, wn_x, wn_tok, bn, token)


def _n3gb_call(agg, h2, t1, wn_agg, wn_x, wn_tok, bn, token, gamma):
    sA, sB = _part_specs(H)
    return pl.pallas_call(
        _n3gb_body,
        grid=(GRID_N,),
        in_specs=[sA, sB, _node_spec(H), _node_spec(16), _wfull((H, 2)),
                  _wfull((H, 2)), _wfull((H, 2)), _wfull((1, 2)),
                  _wfull((1, H)), _wfull((1, 1))],
        out_specs=[_node_spec(16), _SCALAR_SPEC],
        out_shape=[_nshape(16), _SCALAR_SHAPE],
    )(agg, agg, h2, t1, wn_agg, wn_x, wn_tok, bn, token, gamma)


def _gbedge_call(tbs, tbd, geo):
    return pl.pallas_call(
        _gbedge_body,
        grid=(GRID_E,),
        in_specs=[_edge_spec(16), _edge_spec(16), _edge_spec(8)],
        out_specs=[_SCALAR_SPEC, _edge_spec(1), _edge_spec(16), _edge_spec(16)],
        out_shape=[_SCALAR_SHAPE, _eshape(1), _eshape(16), _eshape(16)],
    )(tbs, tbd, geo)


def _gbnode_call(tb, sb, wn3_aggT, wn3_xT, gamma):
    sA, sB = _part_specs(16)
    return pl.pallas_call(
        _gbnode_body,
        grid=(GRID_N,),
        in_specs=[_node_spec(16), sA, sB, _wfull((2, H)), _wfull((2, H)),
                  _wfull((1, 1))],
        out_specs=[_node_spec(H), _node_spec(H)],
        out_shape=[_nshape(H), _nshape(H)],
    )(tb, sb, sb, wn3_aggT, wn3_xT, gamma)


def _bedge3_call(gg, a, wmsT, wmdT, wmkT):
    return pl.pallas_call(
        _bedge3_body,
        grid=(GRID_E,),
        in_specs=[_edge_spec(H), _edge_spec(H), _wfull((H, H)), _wfull((H, H)),
                  _wfull((H, NK))],
        out_specs=[_edge_spec(H), _edge_spec(H), _edge_spec(NK)],
        out_shape=[_eshape(H), _eshape(H), _eshape(NK)],
    )(gg, a, wmsT, wmdT, wmkT)


def _bedge2_call(gg, a, gek_in, wmsT, wmdT, wmkT):
    return pl.pallas_call(
        _bedge2_body,
        grid=(GRID_E,),
        in_specs=[_edge_spec(H), _edge_spec(H), _edge_spec(NK), _wfull((H, H)),
                  _wfull((H, H)), _wfull((H, NK))],
        out_specs=[_edge_spec(H), _edge_spec(H), _edge_spec(NK)],
        out_shape=[_eshape(H), _eshape(H), _eshape(NK)],
    )(gg, a, gek_in, wmsT, wmdT, wmkT)


def _bnode_call(gxdir, sx, hp, wn_aggT, wn_xT):
    sA, sB = _part_specs(H)
    return pl.pallas_call(
        _bnode_body,
        grid=(GRID_N,),
        in_specs=[_node_spec(H), sA, sB, _node_spec(H), _wfull((H, H)),
                  _wfull((H, H))],
        out_specs=[_node_spec(H), _node_spec(H)],
        out_shape=[_nshape(H), _nshape(H)],
    )(gxdir, sx, sx, hp, wn_aggT, wn_xT)


def _b1node_call(gxdir, sx, hp, wn_aggT):
    sA, sB = _part_specs(H)
    return pl.pallas_call(
        _b1node_body,
        grid=(GRID_N,),
        in_specs=[_node_spec(H), sA, sB, _node_spec(H), _wfull((H, H))],
        out_specs=_node_spec(H),
        out_shape=_nshape(H),
    )(gxdir, sx, sx, hp, wn_aggT)


def _bfinal_call(gg1, a1, gek23, geo, gdgb, wm1kT):
    return pl.pallas_call(
        _bfinal_body,
        grid=(GRID_E,),
        in_specs=[_edge_spec(H), _edge_spec(H), _edge_spec(NK), _edge_spec(8),
                  _edge_spec(1), _wfull((H, NK))],
        out_specs=[_edge_spec(16), _edge_spec(16)],
        out_shape=[_eshape(16), _eshape(16)],
    )(gg1, a1, gek23, geo, gdgb, wm1kT)


def _fadd_call(sf):
    sA, sB = _part_specs(16)
    return pl.pallas_call(
        _fadd_body,
        grid=(GRID_N,),
        in_specs=[sA, sB],
        out_specs=_node_spec(16),
        out_shape=_nshape(16),
    )(sf, sf)


_KF = 4
_CH = 128
_GRP = _KF * _CH
_NW = 32
_RPT = ACC_N // 16


def _sc_mesh():
    return plsc.VectorSubcoreMesh(core_axis_name="c", subcore_axis_name="s")


_SC_PARAMS = pltpu.CompilerParams(use_tc_tiling_on_sc=False)


def _gather_scratch(k, w):
    return [pltpu.VMEM((k, _CH), jnp.int32),
            pltpu.VMEM((_GRP, w), jnp.float32),
            pltpu.VMEM((_GRP, w), jnp.float32),
            pltpu.SemaphoreType.DMA,
            pltpu.SemaphoreType.DMA,
            pltpu.SemaphoreType.DMA]


def _gather_worker(table, idx_hbm, out_hbm, idx_v, bufs, semg, semos, lw, k):
    rw = k * _CH
    ng = k // _KF
    pltpu.sync_copy(idx_hbm.at[pl.ds(lw * k, k)], idx_v)

    def pair(go, _):
        for p in (0, 1):
            g = go * 2 + p

            @pl.when(go > 0)
            def _():
                pltpu.make_async_copy(bufs[p], out_hbm.at[pl.ds(0, _GRP)],
                                      semos[p]).wait()

            descs = [
                pltpu.async_copy(table.at[idx_v.at[g * _KF + j]],
                                 bufs[p].at[pl.ds(j * _CH, _CH)], semg)
                for j in range(_KF)
            ]
            for dsc in descs:
                dsc.wait()
            pltpu.async_copy(bufs[p],
                             out_hbm.at[pl.ds(lw * rw + g * _GRP, _GRP)],
                             semos[p])
        return 0

    lax.fori_loop(0, ng // 2, pair, 0)
    for p in (0, 1):
        pltpu.make_async_copy(bufs[p], out_hbm.at[pl.ds(0, _GRP)],
                              semos[p]).wait()


@functools.lru_cache(maxsize=None)
def _mk_gather2(w, nt):
    k = (EPAD // 16) // _CH

    @functools.partial(
        pl.kernel,
        out_type=[jax.ShapeDtypeStruct((EPAD, w), jnp.float32),
                  jax.ShapeDtypeStruct((EPAD, w), jnp.float32)],
        mesh=_sc_mesh(),
        compiler_params=_SC_PARAMS,
        scratch_types=_gather_scratch(k, w),
    )
    def kern(table, idxs, idxd, outs, outd, idx_v, buf0, buf1, semg, so0, so1):
        c = lax.axis_index("c")
        s = lax.axis_index("s")
        wid = s * 2 + c
        lw = wid % 16

        @pl.when(wid < 16)
        def _():
            _gather_worker(table, idxs, outs, idx_v, (buf0, buf1), semg,
                           (so0, so1), lw, k)

        @pl.when(wid >= 16)
        def _():
            _gather_worker(table, idxd, outd, idx_v, (buf0, buf1), semg,
                           (so0, so1), lw, k)

    return kern


@functools.lru_cache(maxsize=None)
def _mk_gather1(w, nt):
    k = (EPAD // _NW) // _CH

    @functools.partial(
        pl.kernel,
        out_type=jax.ShapeDtypeStruct((EPAD, w), jnp.float32),
        mesh=_sc_mesh(),
        compiler_params=_SC_PARAMS,
        scratch_types=_gather_scratch(k, w),
    )
    def kern(table, idx, out, idx_v, buf0, buf1, semg, so0, so1):
        c = lax.axis_index("c")
        s = lax.axis_index("s")
        wid = s * 2 + c
        _gather_worker(table, idx, out, idx_v, (buf0, buf1), semg,
                       (so0, so1), wid, k)

    return kern


def _scatter_worker(vals_hbm, idx_hbm, acc, idx_v, bufs, sema, semis, lw, k):
    rw = k * _CH
    ng = k // _KF
    pltpu.sync_copy(idx_hbm.at[pl.ds(lw * k, k)], idx_v)
    pltpu.async_copy(vals_hbm.at[pl.ds(lw * rw, _GRP)], bufs[0], semis[0])

    def pair(go, _):
        for p in (0, 1):
            g = go * 2 + p
            pltpu.make_async_copy(vals_hbm.at[pl.ds(0, _GRP)], bufs[p],
                                  semis[p]).wait()
            adescs = [
                pltpu.async_copy(bufs[p].at[pl.ds(j * _CH, _CH)],
                                 acc.at[idx_v.at[g * _KF + j]], sema, add=True)
                for j in range(_KF)
            ]

            @pl.when(g + 1 < ng)
            def _():
                pltpu.async_copy(
                    vals_hbm.at[pl.ds(lw * rw + (g + 1) * _GRP, _GRP)],
                    bufs[1 - p], semis[1 - p])

            for dsc in adescs:
                dsc.wait()
        return 0

    lax.fori_loop(0, ng // 2, pair, 0)


def _scatter_scratch(k, w):
    return [pltpu.VMEM((k, _CH), jnp.int32),
            pltpu.VMEM((_GRP, w), jnp.float32),
            pltpu.VMEM((_GRP, w), jnp.float32),
            pltpu.VMEM_SHARED((ACC_N, w), jnp.float32),
            pltpu.SemaphoreType.DMA,
            pltpu.SemaphoreType.DMA,
            pltpu.SemaphoreType.DMA]


@functools.lru_cache(maxsize=None)
def _mk_scatter2(w):
    k = (EPAD // 16) // _CH

    @functools.partial(
        pl.kernel,
        out_type=jax.ShapeDtypeStruct((2, ACC_N, w), jnp.float32),
        mesh=_sc_mesh(),
        compiler_params=_SC_PARAMS,
        scratch_types=_scatter_scratch(k, w),
    )
    def kern(vals_s, vals_d, idxs, idxd, zeros, out, idx_v, buf0, buf1, acc,
             sema, si0, si1):
        c = lax.axis_index("c")
        s = lax.axis_index("s")
        wid = s * 2 + c
        lw = wid % 16
        pltpu.sync_copy(zeros.at[pl.ds(s * _RPT, _RPT)],
                        acc.at[pl.ds(s * _RPT, _RPT)])
        plsc.subcore_barrier()

        @pl.when(wid < 16)
        def _():
            _scatter_worker(vals_s, idxs, acc, idx_v, (buf0, buf1), sema,
                            (si0, si1), lw, k)

        @pl.when(wid >= 16)
        def _():
            _scatter_worker(vals_d, idxd, acc, idx_v, (buf0, buf1), sema,
                            (si0, si1), lw, k)

        plsc.subcore_barrier()
        pltpu.sync_copy(acc.at[pl.ds(s * _RPT, _RPT)],
                        out.at[c, pl.ds(s * _RPT, _RPT)])

    return kern


@functools.lru_cache(maxsize=None)
def _mk_scatter1(w):
    k = (EPAD // _NW) // _CH

    @functools.partial(
        pl.kernel,
        out_type=jax.ShapeDtypeStruct((2, ACC_N, w), jnp.float32),
        mesh=_sc_mesh(),
        compiler_params=_SC_PARAMS,
        scratch_types=_scatter_scratch(k, w),
    )
    def kern(vals, idx, zeros, out, idx_v, buf0, buf1, acc, sema, si0, si1):
        c = lax.axis_index("c")
        s = lax.axis_index("s")
        wid = s * 2 + c
        pltpu.sync_copy(zeros.at[pl.ds(s * _RPT, _RPT)],
                        acc.at[pl.ds(s * _RPT, _RPT)])
        plsc.subcore_barrier()
        _scatter_worker(vals, idx, acc, idx_v, (buf0, buf1), sema,
                        (si0, si1), wid, k)
        plsc.subcore_barrier()
        pltpu.sync_copy(acc.at[pl.ds(s * _RPT, _RPT)],
                        out.at[c, pl.ds(s * _RPT, _RPT)])

    return kern


def _gather2(table, idx_s, idx_d):
    w = table.shape[1]
    return _mk_gather2(w, table.shape[0])(
        table, idx_s.reshape(-1, _CH), idx_d.reshape(-1, _CH))


def _gather1(table, idx):
    w = table.shape[1]
    return _mk_gather1(w, table.shape[0])(table, idx.reshape(-1, _CH))


def _scatter2(vals_s, idx_s, vals_d, idx_d):
    w = vals_s.shape[1]
    zeros = jnp.zeros((ACC_N, w), jnp.float32)
    return _mk_scatter2(w)(vals_s, vals_d, idx_s.reshape(-1, _CH),
                           idx_d.reshape(-1, _CH), zeros)


def _scatter1(vals, idx):
    w = vals.shape[1]
    zeros = jnp.zeros((ACC_N, w), jnp.float32)
    return _mk_scatter1(w)(vals, idx.reshape(-1, _CH), zeros)


def kernel(pos, atom_features, edge_index, solvent_index, emb_solv, gamma_emb,
           Wm1, bm1, Wn1, bn1, Wm2, bm2, Wn2, bn2, Wm3, bm3, Wn3, bn3):
    src = edge_index[0]
    dst = edge_index[1]
    pad_g = jnp.zeros((EPAD - E,), jnp.int32)
    pad_s = jnp.full((EPAD - E,), DUMMY, jnp.int32)
    src_g = jnp.concatenate([src, pad_g])
    dst_g = jnp.concatenate([dst, pad_g])
    src_s = jnp.concatenate([src, pad_s])
    dst_s = jnp.concatenate([dst, pad_s])

    token = emb_solv[solvent_index[0]][None, :]
    gamma = gamma_emb[solvent_index[0], 0].reshape(1, 1)
    t1 = jnp.concatenate([pos, atom_features, jnp.zeros((N, 10), jnp.float32)],
                         axis=1)

    b1 = bm1[None, :]
    b2 = bm2[None, :]
    b3 = bm3[None, :]

    t1s, t1d = _gather2(t1, src_g, dst_g)
    m1, a1, geo = _f1_call(t1s, t1d, Wm1[0:3], Wm1[3:6], Wm1[6:38], b1)
    agg1 = _scatter1(m1, dst_s)
    h1, hp1 = _n1_call(agg1, t1, Wn1[0:64], Wn1[64:67], Wn1[67:131],
                       bn1[None, :], token)

    h1s, h1d = _gather2(h1, src_g, dst_g)
    m2, a2 = _f23_call(h1s, h1d, geo, Wm2[0:64], Wm2[64:128], Wm2[128:160], b2)
    agg2 = _scatter1(m2, dst_s)
    h2, hp2 = _n2_call(agg2, h1, Wn2[0:64], Wn2[64:128], Wn2[128:192],
                       bn2[None, :], token)

    h2s, h2d = _gather2(h2, src_g, dst_g)
    m3, a3 = _f23_call(h2s, h2d, geo, Wm3[0:64], Wm3[64:128], Wm3[128:160], b3)
    agg3 = _scatter1(m3, dst_s)
    tb, es_sum = _n3gb_call(agg3, h2, t1, Wn3[0:64], Wn3[64:128], Wn3[128:192],
                            bn3[None, :], token, gamma)

    tbs, tbd = _gather2(tb, src_g, dst_g)
    ep_sum, gdgb, vsbs, vsbd = _gbedge_call(tbs, tbd, geo)
    sb = _scatter2(vsbs, src_s, vsbd, dst_s)
    gagg3, gxdir3 = _gbnode_call(tb, sb, Wn3[0:64].T, Wn3[64:128].T, gamma)

    gg3 = _gather1(gagg3, dst_g)
    vs3, vd3, gek3 = _bedge3_call(gg3, a3, Wm3[0:64].T, Wm3[64:128].T,
                                  Wm3[128:160].T)
    sx3 = _scatter2(vs3, src_s, vd3, dst_s)
    gagg2, gxdir2 = _bnode_call(gxdir3, sx3, hp2, Wn2[0:64].T, Wn2[64:128].T)

    gg2 = _gather1(gagg2, dst_g)
    vs2, vd2, gek23 = _bedge2_call(gg2, a2, gek3, Wm2[0:64].T, Wm2[64:128].T,
                                   Wm2[128:160].T)
    sx2 = _scatter2(vs2, src_s, vd2, dst_s)
    gagg1 = _b1node_call(gxdir2, sx2, hp1, Wn1[0:64].T)

    gg1 = _gather1(gagg1, dst_g)
    fs, fd = _bfinal_call(gg1, a1, gek23, geo, gdgb, Wm1[6:38].T)
    sf = _scatter2(fs, src_s, fd, dst_s)
    fpad = _fadd_call(sf)

    forces = fpad[:, 0:3]
    energy = (ep_sum + es_sum).reshape(1, 1)
    return energy, forces
